# baseline (speedup 1.0000x reference)
import functools
import jax
import jax.numpy as jnp
from jax import lax
from jax.experimental import pallas as pl
from jax.experimental.pallas import tpu as pltpu
from jax.experimental.pallas import tpu_sc as plsc

N = 100000
E = 1600000
B = 64
HID = 64

NW = 32
EPW = E // NW
ECH = 2000
NCH = ECH // 16
NPAD = 100352
NPW = NPAD // NW

_mesh = plsc.VectorSubcoreMesh(core_axis_name="c", subcore_axis_name="s")


def _wid():
    return lax.axis_index("c") * 16 + lax.axis_index("s")


def _zero_table(table_v, n):
    z = jnp.zeros((16,), jnp.float32)

    def body(j, _):
        table_v[pl.ds(j * 16, 16)] = z
        return _

    lax.fori_loop(0, n // 16, body, None)


def _sck1(dst_hbm, batchf_hbm, degp_hbm, seg_hbm, table_v, idx_v, val_v):
    w = _wid()
    ones = jnp.ones((16,), jnp.float32)

    _zero_table(table_v, NPAD)

    def chunk_a(c, _):
        base = w * EPW + c * ECH
        pltpu.sync_copy(dst_hbm.at[pl.ds(base, ECH)], idx_v)

        def body(j, _):
            idx = idx_v[pl.ds(j * 16, 16)]
            plsc.addupdate_scatter(table_v, [idx], ones)
            return _

        lax.fori_loop(0, NCH, body, None)
        return _

    lax.fori_loop(0, EPW // ECH, chunk_a, None)
    pltpu.sync_copy(table_v, degp_hbm.at[w])

    pltpu.sync_copy(batchf_hbm, table_v)

    def chunk_b(c, _):
        base = w * EPW + c * ECH
        pltpu.sync_copy(dst_hbm.at[pl.ds(base, ECH)], idx_v)

        def body(j, _):
            idx = idx_v[pl.ds(j * 16, 16)]
            val_v[pl.ds(j * 16, 16)] = plsc.load_gather(table_v, [idx])
            return _

        lax.fori_loop(0, NCH, body, None)
        pltpu.sync_copy(val_v, seg_hbm.at[pl.ds(base, ECH)])
        return _

    lax.fori_loop(0, EPW // ECH, chunk_b, None)


_sck1_call = pl.kernel(
    _sck1,
    out_type=(
        jax.ShapeDtypeStruct((NW, NPAD), jnp.float32),
        jax.ShapeDtypeStruct((E,), jnp.float32),
    ),
    mesh=_mesh,
    scratch_types=[
        pltpu.VMEM((NPAD,), jnp.float32),
        pltpu.VMEM((ECH,), jnp.int32),
        pltpu.VMEM((ECH,), jnp.float32),
    ],
    compiler_params=pltpu.CompilerParams(needs_layout_passes=False),
)


def kernel(x, edge_index, batch, dist, sw, W1, b1, W2, b2, W3, b3,
           Wlin, blin, Wlin1, blin1, Wlin2, blin2):
    src = edge_index[0]
    dst = edge_index[1]
    batchf = jnp.zeros((NPAD,), jnp.float32).at[:N].set(batch.astype(jnp.float32))

    degp, seg_f = _sck1_call(dst, batchf)
    deg = jnp.sum(degp, axis=0)[:N] + 1.0
    seg = seg_f.astype(jnp.int32)

    dinv = lax.rsqrt(deg)
    gx = dinv * x[:, 0]
    acc = jnp.zeros((N,), jnp.float32).at[dst].add(gx[src])
    p = dinv * (acc + gx)
    u = jnp.maximum(p, 0.0)
    v = jnp.maximum(-p, 0.0)
    gu = dinv * u
    gv = dinv * v
    accu = jnp.zeros((N,), jnp.float32).at[dst].add(gu[src])
    accv = jnp.zeros((N,), jnp.float32).at[dst].add(gv[src])
    Pu = dinv * (accu + gu)
    Pv = dinv * (accv + gv)

    a = jnp.maximum(W1[0], 0.0)
    c = jnp.maximum(-W1[0], 0.0)
    alpha = a @ W2
    gamma = c @ W2
    ts = jnp.sort(jnp.where(gamma != 0, -alpha / jnp.where(gamma != 0, gamma, 1.0), 0.0))
    r_rep = jnp.concatenate([ts[:1] - 1.0, (ts[:-1] + ts[1:]) * 0.5, ts[-1:] + 1.0])
    M = ((alpha[None, :] + gamma[None, :] * r_rep[:, None]) > 0.0).astype(jnp.float32)

    Pus = Pu[src]
    Pvs = Pv[src]
    norm = dinv[src] * dinv[dst]

    def buckets(pu, pv):
        r = pv / pu
        return jnp.sum(ts[None, :] < r[:, None], axis=1)

    bk_e = buckets(Pus, Pvs)
    bk_n = buckets(Pu, Pv)
    Su = jnp.zeros((B, 65), jnp.float32).at[seg, bk_e].add(norm * Pus)
    Sv = jnp.zeros((B, 65), jnp.float32).at[seg, bk_e].add(norm * Pvs)
    w_self = dinv * dinv
    Su = Su.at[batch, bk_n].add(w_self * Pu)
    Sv = Sv.at[batch, bk_n].add(w_self * Pv)
    cnt = jnp.zeros((B,), jnp.float32).at[batch].add(1.0)

    A = alpha[None, :] * (Su @ M) + gamma[None, :] * (Sv @ M)
    pooled = (A / jnp.maximum(cnt, 1.0)[:, None]) @ W3 + b3
    z = pooled @ Wlin + blin
    z = jnp.concatenate([z, dist, sw], axis=1)
    z = jax.nn.relu(z @ Wlin1 + blin1)
    return z @ Wlin2 + blin2


# speedup vs baseline: 77.1952x; 77.1952x over previous
import jax
import jax.numpy as jnp
from jax import lax
from jax.experimental import pallas as pl
from jax.experimental.pallas import tpu as pltpu
from jax.experimental.pallas import tpu_sc as plsc

N = 100000
E = 1600000
B = 64
HID = 64

NW = 32
EPW = E // NW
ECH = 2000
NCH = ECH // 16
NPAD = 100352
NPW = NPAD // NW
NNCH = NPW // 2
NB = 65
SROW = 784

_mesh = plsc.VectorSubcoreMesh(core_axis_name="c", subcore_axis_name="s")
_sc_params = pltpu.CompilerParams(needs_layout_passes=False)


def _wid():
    return lax.axis_index("c") * 16 + lax.axis_index("s")


def _zero_table(table_v, n):
    z = jnp.zeros((16,), jnp.float32)

    def body(j, carry):
        table_v[pl.ds(j * 16, 16)] = z
        return carry

    lax.fori_loop(0, n // 16, body, None)


def _edge_chunks(w, fn):
    def chunk(c, carry):
        fn(w * EPW + c * ECH)
        return carry

    lax.fori_loop(0, EPW // ECH, chunk, None)


def _sc1(dst_hbm, batchf_hbm, degp_hbm, seg_hbm, table_v, idx_v, idx2_v, val_v):
    w = _wid()
    ones = jnp.ones((16,), jnp.float32)

    _zero_table(table_v, NPAD)

    def deg_chunk(base):
        pltpu.sync_copy(dst_hbm.at[pl.ds(base, ECH)], idx_v)

        def body(j, carry):
            idx = idx_v[pl.ds(j * 16, 16)]
            plsc.addupdate_scatter(table_v, [idx], ones)
            return carry

        lax.fori_loop(0, NCH, body, None)

    _edge_chunks(w, deg_chunk)
    pltpu.sync_copy(table_v, degp_hbm.at[w])

    pltpu.sync_copy(batchf_hbm, table_v)

    def seg_chunk(base):
        pltpu.sync_copy(dst_hbm.at[pl.ds(base, ECH)], idx_v)

        def body(j, carry):
            idx = idx_v[pl.ds(j * 16, 16)]
            val_v[pl.ds(j * 16, 16)] = plsc.load_gather(table_v, [idx])
            return carry

        lax.fori_loop(0, NCH, body, None)
        pltpu.sync_copy(val_v, seg_hbm.at[pl.ds(base, ECH)])

    _edge_chunks(w, seg_chunk)


_sc1_call = pl.kernel(
    _sc1,
    out_type=(
        jax.ShapeDtypeStruct((NW, NPAD), jnp.float32),
        jax.ShapeDtypeStruct((E,), jnp.float32),
    ),
    mesh=_mesh,
    scratch_types=[
        pltpu.VMEM((NPAD,), jnp.float32),
        pltpu.VMEM((ECH,), jnp.int32),
        pltpu.VMEM((ECH,), jnp.int32),
        pltpu.VMEM((ECH,), jnp.float32),
    ],
    compiler_params=_sc_params,
)


def _sc2(src_hbm, dst_hbm, dinv_hbm, gx_hbm, norm_hbm, gxs_hbm,
         paccp_hbm, table_v, idx_v, idx2_v, val_v):
    w = _wid()

    pltpu.sync_copy(dinv_hbm, table_v)

    def norm_chunk(base):
        pltpu.sync_copy(src_hbm.at[pl.ds(base, ECH)], idx_v)
        pltpu.sync_copy(dst_hbm.at[pl.ds(base, ECH)], idx2_v)

        def body(j, carry):
            s = idx_v[pl.ds(j * 16, 16)]
            d = idx2_v[pl.ds(j * 16, 16)]
            ds_ = plsc.load_gather(table_v, [s])
            dd = plsc.load_gather(table_v, [d])
            val_v[pl.ds(j * 16, 16)] = ds_ * dd
            return carry

        lax.fori_loop(0, NCH, body, None)
        pltpu.sync_copy(val_v, norm_hbm.at[pl.ds(base, ECH)])

    _edge_chunks(w, norm_chunk)

    pltpu.sync_copy(gx_hbm, table_v)

    def gxs_chunk(base):
        pltpu.sync_copy(src_hbm.at[pl.ds(base, ECH)], idx_v)

        def body(j, carry):
            s = idx_v[pl.ds(j * 16, 16)]
            val_v[pl.ds(j * 16, 16)] = plsc.load_gather(table_v, [s])
            return carry

        lax.fori_loop(0, NCH, body, None)
        pltpu.sync_copy(val_v, gxs_hbm.at[pl.ds(base, ECH)])

    _edge_chunks(w, gxs_chunk)

    _zero_table(table_v, NPAD)

    def pacc_chunk(base):
        pltpu.sync_copy(dst_hbm.at[pl.ds(base, ECH)], idx_v)
        pltpu.sync_copy(gxs_hbm.at[pl.ds(base, ECH)], val_v)

        def body(j, carry):
            d = idx_v[pl.ds(j * 16, 16)]
            x = val_v[pl.ds(j * 16, 16)]
            plsc.addupdate_scatter(table_v, [d], x)
            return carry

        lax.fori_loop(0, NCH, body, None)

    _edge_chunks(w, pacc_chunk)
    pltpu.sync_copy(table_v, paccp_hbm.at[w])


_sc2_call = pl.kernel(
    _sc2,
    out_type=(
        jax.ShapeDtypeStruct((E,), jnp.float32),
        jax.ShapeDtypeStruct((E,), jnp.float32),
        jax.ShapeDtypeStruct((NW, NPAD), jnp.float32),
    ),
    mesh=_mesh,
    scratch_types=[
        pltpu.VMEM((NPAD,), jnp.float32),
        pltpu.VMEM((ECH,), jnp.int32),
        pltpu.VMEM((ECH,), jnp.int32),
        pltpu.VMEM((ECH,), jnp.float32),
    ],
    compiler_params=_sc_params,
)


def _sc3(src_hbm, dst_hbm, gu_hbm, gv_hbm, gus_hbm, gvs_hbm,
         uaccp_hbm, vaccp_hbm, table_v, idx_v, idx2_v, val_v):
    w = _wid()

    def gather_phase(tab_hbm, out_hbm):
        pltpu.sync_copy(tab_hbm, table_v)

        def g_chunk(base):
            pltpu.sync_copy(src_hbm.at[pl.ds(base, ECH)], idx_v)

            def body(j, carry):
                s = idx_v[pl.ds(j * 16, 16)]
                val_v[pl.ds(j * 16, 16)] = plsc.load_gather(table_v, [s])
                return carry

            lax.fori_loop(0, NCH, body, None)
            pltpu.sync_copy(val_v, out_hbm.at[pl.ds(base, ECH)])

        _edge_chunks(w, g_chunk)

    def scatter_phase(vals_hbm, out_hbm):
        _zero_table(table_v, NPAD)

        def s_chunk(base):
            pltpu.sync_copy(dst_hbm.at[pl.ds(base, ECH)], idx_v)
            pltpu.sync_copy(vals_hbm.at[pl.ds(base, ECH)], val_v)

            def body(j, carry):
                d = idx_v[pl.ds(j * 16, 16)]
                x = val_v[pl.ds(j * 16, 16)]
                plsc.addupdate_scatter(table_v, [d], x)
                return carry

            lax.fori_loop(0, NCH, body, None)

        _edge_chunks(w, s_chunk)
        pltpu.sync_copy(table_v, out_hbm.at[w])

    gather_phase(gu_hbm, gus_hbm)
    gather_phase(gv_hbm, gvs_hbm)
    scatter_phase(gus_hbm, uaccp_hbm)
    scatter_phase(gvs_hbm, vaccp_hbm)


_sc3_call = pl.kernel(
    _sc3,
    out_type=(
        jax.ShapeDtypeStruct((E,), jnp.float32),
        jax.ShapeDtypeStruct((E,), jnp.float32),
        jax.ShapeDtypeStruct((NW, NPAD), jnp.float32),
        jax.ShapeDtypeStruct((NW, NPAD), jnp.float32),
    ),
    mesh=_mesh,
    scratch_types=[
        pltpu.VMEM((NPAD,), jnp.float32),
        pltpu.VMEM((ECH,), jnp.int32),
        pltpu.VMEM((ECH,), jnp.int32),
        pltpu.VMEM((ECH,), jnp.float32),
    ],
    compiler_params=_sc_params,
)


def _sc4(src_hbm, pu_hbm, pv_hbm, norm_hbm, seg_hbm, wslf_hbm, batchf_hbm,
         sn_hbm, ts_hbm, pus_hbm, pvs_hbm, sup_hbm, svp_hbm, cntp_hbm,
         table_v, idx_v, pu_v, pv_v, wt_v, seg_v, sn_v, ts_v, su_v, sv_v,
         cnt_v):
    w = _wid()

    def gather_phase(tab_hbm, out_hbm):
        pltpu.sync_copy(tab_hbm, table_v)

        def g_chunk(base):
            pltpu.sync_copy(src_hbm.at[pl.ds(base, ECH)], idx_v)

            def body(j, carry):
                s = idx_v[pl.ds(j * 16, 16)]
                pu_v[pl.ds(j * 16, 16)] = plsc.load_gather(table_v, [s])
                return carry

            lax.fori_loop(0, NCH, body, None)
            pltpu.sync_copy(pu_v, out_hbm.at[pl.ds(base, ECH)])

        _edge_chunks(w, g_chunk)

    gather_phase(pu_hbm, pus_hbm)
    gather_phase(pv_hbm, pvs_hbm)

    pltpu.sync_copy(ts_hbm, ts_v)
    _zero_table(su_v, B * NB)
    _zero_table(sv_v, B * NB)
    _zero_table(cnt_v, B)

    def fold_vreg(j):
        pu = pu_v[pl.ds(j * 16, 16)]
        pv = pv_v[pl.ds(j * 16, 16)]
        wt = wt_v[pl.ds(j * 16, 16)]
        sg = seg_v[pl.ds(j * 16, 16)].astype(jnp.int32)
        r = pv / pu
        lo = jnp.zeros((16,), jnp.int32)
        for step in (64, 32, 16, 8, 4, 2, 1):
            probe = lo + (step - 1)
            t = plsc.load_gather(ts_v, [probe])
            lo = lo + jnp.where(t < r, step, 0)
        flat = sg * NB + lo
        plsc.addupdate_scatter(su_v, [flat], wt * pu)
        plsc.addupdate_scatter(sv_v, [flat], wt * pv)

    def edge_fold_chunk(base):
        pltpu.sync_copy(pus_hbm.at[pl.ds(base, ECH)], pu_v)
        pltpu.sync_copy(pvs_hbm.at[pl.ds(base, ECH)], pv_v)
        pltpu.sync_copy(norm_hbm.at[pl.ds(base, ECH)], wt_v)
        pltpu.sync_copy(seg_hbm.at[pl.ds(base, ECH)], seg_v)

        def body(j, carry):
            fold_vreg(j)
            return carry

        lax.fori_loop(0, NCH, body, None)

    _edge_chunks(w, edge_fold_chunk)

    ones = jnp.ones((16,), jnp.float32)

    def node_chunk(c, carry):
        base = w * NPW + c * NNCH
        pltpu.sync_copy(pu_hbm.at[pl.ds(base, NNCH)], pu_v.at[pl.ds(0, NNCH)])
        pltpu.sync_copy(pv_hbm.at[pl.ds(base, NNCH)], pv_v.at[pl.ds(0, NNCH)])
        pltpu.sync_copy(wslf_hbm.at[pl.ds(base, NNCH)], wt_v.at[pl.ds(0, NNCH)])
        pltpu.sync_copy(batchf_hbm.at[pl.ds(base, NNCH)], seg_v.at[pl.ds(0, NNCH)])
        pltpu.sync_copy(sn_hbm.at[pl.ds(base, NNCH)], sn_v.at[pl.ds(0, NNCH)])

        def body(j, carry2):
            fold_vreg(j)
            sg = seg_v[pl.ds(j * 16, 16)].astype(jnp.int32)
            plsc.addupdate_scatter(cnt_v, [sg], sn_v[pl.ds(j * 16, 16)])
            return carry2

        lax.fori_loop(0, NNCH // 16, body, None)
        return carry

    lax.fori_loop(0, 2, node_chunk, None)

    pltpu.sync_copy(su_v, sup_hbm.at[w])
    pltpu.sync_copy(sv_v, svp_hbm.at[w])
    pltpu.sync_copy(cnt_v, cntp_hbm.at[w])


_sc4_call = pl.kernel(
    _sc4,
    out_type=(
        jax.ShapeDtypeStruct((E,), jnp.float32),
        jax.ShapeDtypeStruct((E,), jnp.float32),
        jax.ShapeDtypeStruct((NW, B * NB), jnp.float32),
        jax.ShapeDtypeStruct((NW, B * NB), jnp.float32),
        jax.ShapeDtypeStruct((NW, B), jnp.float32),
    ),
    mesh=_mesh,
    scratch_types=[
        pltpu.VMEM((NPAD,), jnp.float32),
        pltpu.VMEM((ECH,), jnp.int32),
        pltpu.VMEM((ECH,), jnp.float32),
        pltpu.VMEM((ECH,), jnp.float32),
        pltpu.VMEM((ECH,), jnp.float32),
        pltpu.VMEM((ECH,), jnp.float32),
        pltpu.VMEM((ECH,), jnp.float32),
        pltpu.VMEM((128,), jnp.float32),
        pltpu.VMEM((B * NB,), jnp.float32),
        pltpu.VMEM((B * NB,), jnp.float32),
        pltpu.VMEM((B,), jnp.float32),
    ],
    compiler_params=_sc_params,
)


def _tc1(degp_ref, x_ref, dinv_ref, gx_ref, w_ref):
    deg = jnp.sum(degp_ref[...], axis=0) + 1.0
    dinv = lax.rsqrt(deg)
    dinv_ref[...] = dinv
    gx_ref[...] = dinv * x_ref[...]
    w_ref[...] = dinv * dinv


def _tc2(paccp_ref, gx_ref, dinv_ref, gu_ref, gv_ref):
    acc = jnp.sum(paccp_ref[...], axis=0)
    dinv = dinv_ref[...]
    p = dinv * (acc + gx_ref[...])
    gu_ref[...] = dinv * jnp.maximum(p, 0.0)
    gv_ref[...] = dinv * jnp.maximum(-p, 0.0)


def _tc3(uaccp_ref, vaccp_ref, gu_ref, gv_ref, dinv_ref, pu_ref, pv_ref):
    dinv = dinv_ref[...]
    pu_ref[...] = dinv * (jnp.sum(uaccp_ref[...], axis=0) + gu_ref[...])
    pv_ref[...] = dinv * (jnp.sum(vaccp_ref[...], axis=0) + gv_ref[...])


def _tc4(sup_ref, svp_ref, cntp_ref, m_ref, al_ref, ga_ref, w3_ref, b3_ref,
         wl_ref, bl_ref, ds_ref, sw_ref, wl1_ref, bl1_ref, wl2_ref, bl2_ref,
         out_ref):
    su = jnp.sum(sup_ref[...], axis=0)
    sv = jnp.sum(svp_ref[...], axis=0)
    cnt = jnp.maximum(jnp.sum(cntp_ref[...], axis=0), 1.0)
    m = m_ref[...]
    a_mat = (al_ref[...] * jnp.dot(su, m, preferred_element_type=jnp.float32)
             + ga_ref[...] * jnp.dot(sv, m, preferred_element_type=jnp.float32))
    pooled = jnp.dot(a_mat / cnt[:, None], w3_ref[...],
                     preferred_element_type=jnp.float32) + b3_ref[...]
    z = jnp.dot(pooled, wl_ref[...],
                preferred_element_type=jnp.float32) + bl_ref[...]
    z = jnp.concatenate([z, ds_ref[...], sw_ref[...]], axis=1)
    z = jnp.maximum(jnp.dot(z, wl1_ref[...],
                            preferred_element_type=jnp.float32) + bl1_ref[...], 0.0)
    out_ref[...] = jnp.dot(z, wl2_ref[...],
                           preferred_element_type=jnp.float32) + bl2_ref[...]


def _tc_call(fn, out_shapes):
    return pl.pallas_call(fn, out_shape=out_shapes)


def kernel(x, edge_index, batch, dist, sw, W1, b1, W2, b2, W3, b3,
           Wlin, blin, Wlin1, blin1, Wlin2, blin2):
    f32 = jnp.float32
    src = edge_index[0]
    dst = edge_index[1]
    batchf = jnp.zeros((NPAD,), f32).at[:N].set(batch.astype(f32))
    xpad = jnp.zeros((NPAD,), f32).at[:N].set(x[:, 0])
    sn = jnp.zeros((NPAD,), f32).at[:N].set(1.0)

    a = jnp.maximum(W1[0], 0.0)
    c = jnp.maximum(-W1[0], 0.0)
    alpha = a @ W2
    gamma = c @ W2
    ts = jnp.sort(jnp.where(gamma != 0, -alpha / jnp.where(gamma != 0, gamma, 1.0), 0.0))
    ts_pad = jnp.concatenate([ts, jnp.full((64,), jnp.inf, f32)])
    r_rep = jnp.concatenate([ts[:1] - 1.0, (ts[:-1] + ts[1:]) * 0.5, ts[-1:] + 1.0])
    M = ((alpha[None, :] + gamma[None, :] * r_rep[:, None]) > 0.0).astype(f32)

    degp, seg_f = _sc1_call(dst, batchf)

    nshape = jax.ShapeDtypeStruct((SROW, 128), f32)
    dinv2d, gx2d, w2d = _tc_call(_tc1, (nshape, nshape, nshape))(
        degp.reshape(NW, SROW, 128), xpad.reshape(SROW, 128))
    dinv = dinv2d.reshape(NPAD)
    gx = gx2d.reshape(NPAD)
    wslf = w2d.reshape(NPAD)

    norm, gxs, paccp = _sc2_call(src, dst, dinv, gx)

    gu2d, gv2d = _tc_call(_tc2, (nshape, nshape))(
        paccp.reshape(NW, SROW, 128), gx2d, dinv2d)
    gu = gu2d.reshape(NPAD)
    gv = gv2d.reshape(NPAD)

    _, _, uaccp, vaccp = _sc3_call(src, dst, gu, gv)

    pu2d, pv2d = _tc_call(_tc3, (nshape, nshape))(
        uaccp.reshape(NW, SROW, 128), vaccp.reshape(NW, SROW, 128),
        gu2d, gv2d, dinv2d)
    pu = pu2d.reshape(NPAD)
    pv = pv2d.reshape(NPAD)

    _, _, sup, svp, cntp = _sc4_call(src, pu, pv, norm, seg_f, wslf, batchf,
                                     sn, ts_pad)

    out = _tc_call(_tc4, jax.ShapeDtypeStruct((B, 1), f32))(
        sup.reshape(NW, B, NB), svp.reshape(NW, B, NB), cntp, M,
        alpha[None, :], gamma[None, :], W3, b3[None, :],
        Wlin, blin[None, :], dist, sw, Wlin1, blin1[None, :], Wlin2,
        blin2[None, :])
    return out


# speedup vs baseline: 111.2310x; 1.4409x over previous
import jax
import jax.numpy as jnp
from jax import lax
from jax.experimental import pallas as pl
from jax.experimental.pallas import tpu as pltpu
from jax.experimental.pallas import tpu_sc as plsc

N = 100000
E = 1600000
B = 64
HID = 64

NW = 32
EPW = E // NW
ECH = 2000
NCH = ECH // 16
NPAD = 100352
NPW = NPAD // NW
NNCH = NPW // 2
NB = 65
SUSZ = 4240
SROW = 784
UNROLL = 8

_mesh = plsc.VectorSubcoreMesh(core_axis_name="c", subcore_axis_name="s")
_sc_params = pltpu.CompilerParams(needs_layout_passes=False)


def _wid():
    return lax.axis_index("c") * 16 + lax.axis_index("s")


def _zero_table(table_v, n):
    z = jnp.zeros((16,), jnp.float32)

    @plsc.parallel_loop(0, n // 16, unroll=UNROLL)
    def _(j):
        table_v[pl.ds(j * 16, 16)] = z


def _edge_chunks(w, fn):
    def chunk(c, carry):
        fn(w * EPW + c * ECH)
        return carry

    lax.fori_loop(0, EPW // ECH, chunk, None)


def _sc1(dst_hbm, degp_hbm, table_v, idx_v, val_v):
    w = _wid()
    ones = jnp.ones((16,), jnp.float32)

    _zero_table(table_v, NPAD)

    def deg_chunk(base):
        pltpu.sync_copy(dst_hbm.at[pl.ds(base, ECH)], idx_v)

        @plsc.parallel_loop(0, NCH, unroll=UNROLL)
        def _(j):
            idx = idx_v[pl.ds(j * 16, 16)]
            plsc.addupdate_scatter(table_v, [idx], ones)

    _edge_chunks(w, deg_chunk)
    pltpu.sync_copy(table_v, degp_hbm.at[w])


_sc1_call = pl.kernel(
    _sc1,
    out_type=jax.ShapeDtypeStruct((NW, NPAD), jnp.float32),
    mesh=_mesh,
    scratch_types=[
        pltpu.VMEM((NPAD,), jnp.float32),
        pltpu.VMEM((ECH,), jnp.int32),
        pltpu.VMEM((ECH,), jnp.float32),
    ],
    compiler_params=_sc_params,
)


def _sc2(src_hbm, dst_hbm, dinv_hbm, gx_hbm, norm_hbm, gxs_hbm,
         paccp_hbm, table_v, idx_v, idx2_v, val_v):
    w = _wid()

    pltpu.sync_copy(dinv_hbm, table_v)

    def norm_chunk(base):
        pltpu.sync_copy(src_hbm.at[pl.ds(base, ECH)], idx_v)
        pltpu.sync_copy(dst_hbm.at[pl.ds(base, ECH)], idx2_v)

        @plsc.parallel_loop(0, NCH, unroll=UNROLL)
        def _(j):
            s = idx_v[pl.ds(j * 16, 16)]
            d = idx2_v[pl.ds(j * 16, 16)]
            val_v[pl.ds(j * 16, 16)] = (plsc.load_gather(table_v, [s])
                                        * plsc.load_gather(table_v, [d]))

        pltpu.sync_copy(val_v, norm_hbm.at[pl.ds(base, ECH)])

    _edge_chunks(w, norm_chunk)

    pltpu.sync_copy(gx_hbm, table_v)

    def gxs_chunk(base):
        pltpu.sync_copy(src_hbm.at[pl.ds(base, ECH)], idx_v)

        @plsc.parallel_loop(0, NCH, unroll=UNROLL)
        def _(j):
            s = idx_v[pl.ds(j * 16, 16)]
            val_v[pl.ds(j * 16, 16)] = plsc.load_gather(table_v, [s])

        pltpu.sync_copy(val_v, gxs_hbm.at[pl.ds(base, ECH)])

    _edge_chunks(w, gxs_chunk)

    _zero_table(table_v, NPAD)

    def pacc_chunk(base):
        pltpu.sync_copy(dst_hbm.at[pl.ds(base, ECH)], idx_v)
        pltpu.sync_copy(gxs_hbm.at[pl.ds(base, ECH)], val_v)

        @plsc.parallel_loop(0, NCH, unroll=UNROLL)
        def _(j):
            d = idx_v[pl.ds(j * 16, 16)]
            plsc.addupdate_scatter(table_v, [d], val_v[pl.ds(j * 16, 16)])

    _edge_chunks(w, pacc_chunk)
    pltpu.sync_copy(table_v, paccp_hbm.at[w])


_sc2_call = pl.kernel(
    _sc2,
    out_type=(
        jax.ShapeDtypeStruct((E,), jnp.float32),
        jax.ShapeDtypeStruct((E,), jnp.float32),
        jax.ShapeDtypeStruct((NW, NPAD), jnp.float32),
    ),
    mesh=_mesh,
    scratch_types=[
        pltpu.VMEM((NPAD,), jnp.float32),
        pltpu.VMEM((ECH,), jnp.int32),
        pltpu.VMEM((ECH,), jnp.int32),
        pltpu.VMEM((ECH,), jnp.float32),
    ],
    compiler_params=_sc_params,
)


def _sc3(src_hbm, dst_hbm, gu_hbm, gv_hbm, gus_hbm, gvs_hbm,
         uaccp_hbm, vaccp_hbm, table_v, idx_v, val_v):
    w = _wid()

    def gather_phase(tab_hbm, out_hbm):
        pltpu.sync_copy(tab_hbm, table_v)

        def g_chunk(base):
            pltpu.sync_copy(src_hbm.at[pl.ds(base, ECH)], idx_v)

            @plsc.parallel_loop(0, NCH, unroll=UNROLL)
            def _(j):
                s = idx_v[pl.ds(j * 16, 16)]
                val_v[pl.ds(j * 16, 16)] = plsc.load_gather(table_v, [s])

            pltpu.sync_copy(val_v, out_hbm.at[pl.ds(base, ECH)])

        _edge_chunks(w, g_chunk)

    def scatter_phase(vals_hbm, out_hbm):
        _zero_table(table_v, NPAD)

        def s_chunk(base):
            pltpu.sync_copy(dst_hbm.at[pl.ds(base, ECH)], idx_v)
            pltpu.sync_copy(vals_hbm.at[pl.ds(base, ECH)], val_v)

            @plsc.parallel_loop(0, NCH, unroll=UNROLL)
            def _(j):
                d = idx_v[pl.ds(j * 16, 16)]
                plsc.addupdate_scatter(table_v, [d], val_v[pl.ds(j * 16, 16)])

        _edge_chunks(w, s_chunk)
        pltpu.sync_copy(table_v, out_hbm.at[w])

    gather_phase(gu_hbm, gus_hbm)
    gather_phase(gv_hbm, gvs_hbm)
    scatter_phase(gus_hbm, uaccp_hbm)
    scatter_phase(gvs_hbm, vaccp_hbm)


_sc3_call = pl.kernel(
    _sc3,
    out_type=(
        jax.ShapeDtypeStruct((E,), jnp.float32),
        jax.ShapeDtypeStruct((E,), jnp.float32),
        jax.ShapeDtypeStruct((NW, NPAD), jnp.float32),
        jax.ShapeDtypeStruct((NW, NPAD), jnp.float32),
    ),
    mesh=_mesh,
    scratch_types=[
        pltpu.VMEM((NPAD,), jnp.float32),
        pltpu.VMEM((ECH,), jnp.int32),
        pltpu.VMEM((ECH,), jnp.float32),
    ],
    compiler_params=_sc_params,
)


def _sc4(src_hbm, dst_hbm, t_hbm, pu_hbm, pv_hbm, norm_hbm, wslf_hbm,
         flat_hbm, sup_hbm, svp_hbm, cntp_hbm,
         table_v, idx_v, idx2_v, flat_v, norm_v, val_v, ws_v, su_v, sv_v,
         cnt_v):
    w = _wid()

    pltpu.sync_copy(t_hbm, table_v)

    def flat_chunk(base):
        pltpu.sync_copy(src_hbm.at[pl.ds(base, ECH)], idx_v)
        pltpu.sync_copy(dst_hbm.at[pl.ds(base, ECH)], idx2_v)

        @plsc.parallel_loop(0, NCH, unroll=UNROLL)
        def _(j):
            s = idx_v[pl.ds(j * 16, 16)]
            d = idx2_v[pl.ds(j * 16, 16)]
            ts_ = plsc.load_gather(table_v, [s]).astype(jnp.int32)
            td = plsc.load_gather(table_v, [d]).astype(jnp.int32)
            flat = (td >> 8) * NB + (ts_ & 255)
            flat_v[pl.ds(j * 16, 16)] = flat.astype(jnp.float32)

        pltpu.sync_copy(flat_v, flat_hbm.at[pl.ds(base, ECH)])

    _edge_chunks(w, flat_chunk)

    _zero_table(su_v, SUSZ)
    _zero_table(sv_v, SUSZ)
    _zero_table(cnt_v, 80)

    def fold_phase(tab_hbm, acc_v):
        pltpu.sync_copy(tab_hbm, table_v)

        def f_chunk(base):
            pltpu.sync_copy(src_hbm.at[pl.ds(base, ECH)], idx_v)
            pltpu.sync_copy(flat_hbm.at[pl.ds(base, ECH)], flat_v)
            pltpu.sync_copy(norm_hbm.at[pl.ds(base, ECH)], norm_v)

            @plsc.parallel_loop(0, NCH, unroll=UNROLL)
            def _(j):
                s = idx_v[pl.ds(j * 16, 16)]
                g = plsc.load_gather(table_v, [s])
                fl = flat_v[pl.ds(j * 16, 16)].astype(jnp.int32)
                nm = norm_v[pl.ds(j * 16, 16)]
                plsc.addupdate_scatter(acc_v, [fl], nm * g)

        _edge_chunks(w, f_chunk)

    fold_phase(pu_hbm, su_v)
    fold_phase(pv_hbm, sv_v)

    ones = jnp.ones((16,), jnp.float32)

    def node_chunk(c, carry):
        base = w * NPW + c * NNCH
        pltpu.sync_copy(t_hbm.at[pl.ds(base, NNCH)], flat_v.at[pl.ds(0, NNCH)])
        pltpu.sync_copy(pu_hbm.at[pl.ds(base, NNCH)], val_v.at[pl.ds(0, NNCH)])
        pltpu.sync_copy(pv_hbm.at[pl.ds(base, NNCH)], norm_v.at[pl.ds(0, NNCH)])
        pltpu.sync_copy(wslf_hbm.at[pl.ds(base, NNCH)], ws_v.at[pl.ds(0, NNCH)])

        @plsc.parallel_loop(0, NNCH // 16, unroll=UNROLL)
        def _(j):
            t = flat_v[pl.ds(j * 16, 16)].astype(jnp.int32)
            sg = t >> 8
            fl = sg * NB + (t & 255)
            wt = ws_v[pl.ds(j * 16, 16)]
            plsc.addupdate_scatter(su_v, [fl], wt * val_v[pl.ds(j * 16, 16)])
            plsc.addupdate_scatter(sv_v, [fl], wt * norm_v[pl.ds(j * 16, 16)])
            plsc.addupdate_scatter(cnt_v, [sg], ones)

        return carry

    lax.fori_loop(0, 2, node_chunk, None)

    pltpu.sync_copy(su_v, sup_hbm.at[w])
    pltpu.sync_copy(sv_v, svp_hbm.at[w])
    pltpu.sync_copy(cnt_v, cntp_hbm.at[w])


_sc4_call = pl.kernel(
    _sc4,
    out_type=(
        jax.ShapeDtypeStruct((E,), jnp.float32),
        jax.ShapeDtypeStruct((NW, SUSZ), jnp.float32),
        jax.ShapeDtypeStruct((NW, SUSZ), jnp.float32),
        jax.ShapeDtypeStruct((NW, 80), jnp.float32),
    ),
    mesh=_mesh,
    scratch_types=[
        pltpu.VMEM((NPAD,), jnp.float32),
        pltpu.VMEM((ECH,), jnp.int32),
        pltpu.VMEM((ECH,), jnp.int32),
        pltpu.VMEM((ECH,), jnp.float32),
        pltpu.VMEM((ECH,), jnp.float32),
        pltpu.VMEM((ECH,), jnp.float32),
        pltpu.VMEM((ECH,), jnp.float32),
        pltpu.VMEM((SUSZ,), jnp.float32),
        pltpu.VMEM((SUSZ,), jnp.float32),
        pltpu.VMEM((80,), jnp.float32),
    ],
    compiler_params=_sc_params,
)


def _tc1(degp_ref, x_ref, dinv_ref, gx_ref, w_ref):
    deg = jnp.sum(degp_ref[...], axis=0) + 1.0
    dinv = lax.rsqrt(deg)
    dinv_ref[...] = dinv
    gx_ref[...] = dinv * x_ref[...]
    w_ref[...] = dinv * dinv


def _tc2(paccp_ref, gx_ref, dinv_ref, gu_ref, gv_ref):
    acc = jnp.sum(paccp_ref[...], axis=0)
    dinv = dinv_ref[...]
    p = dinv * (acc + gx_ref[...])
    gu_ref[...] = dinv * jnp.maximum(p, 0.0)
    gv_ref[...] = dinv * jnp.maximum(-p, 0.0)


def _tc3(uaccp_ref, vaccp_ref, gu_ref, gv_ref, dinv_ref, batchf_ref, ts_ref,
         pu_ref, pv_ref, t_ref):
    dinv = dinv_ref[...]
    pu = dinv * (jnp.sum(uaccp_ref[...], axis=0) + gu_ref[...])
    pv = dinv * (jnp.sum(vaccp_ref[...], axis=0) + gv_ref[...])
    pu_ref[...] = pu
    pv_ref[...] = pv
    r = pv / pu
    bk = jnp.sum((ts_ref[...][0][None, None, :] < r[:, :, None]).astype(jnp.float32),
                 axis=-1)
    t_ref[...] = batchf_ref[...] * 256.0 + bk


def _tc4(sup_ref, svp_ref, cntp_ref, m_ref, al_ref, ga_ref, w3_ref, b3_ref,
         wl_ref, bl_ref, ds_ref, sw_ref, wl1_ref, bl1_ref, wl2_ref, bl2_ref,
         out_ref):
    su = jnp.sum(sup_ref[...], axis=0)
    sv = jnp.sum(svp_ref[...], axis=0)
    cnt = jnp.maximum(jnp.sum(cntp_ref[...], axis=0), 1.0)
    m = m_ref[...]
    a_mat = (al_ref[...] * jnp.dot(su, m, preferred_element_type=jnp.float32)
             + ga_ref[...] * jnp.dot(sv, m, preferred_element_type=jnp.float32))
    pooled = jnp.dot(a_mat / cnt[:, None], w3_ref[...],
                     preferred_element_type=jnp.float32) + b3_ref[...]
    z = jnp.dot(pooled, wl_ref[...],
                preferred_element_type=jnp.float32) + bl_ref[...]
    z = jnp.concatenate([z, ds_ref[...], sw_ref[...]], axis=1)
    z = jnp.maximum(jnp.dot(z, wl1_ref[...],
                            preferred_element_type=jnp.float32) + bl1_ref[...], 0.0)
    out_ref[...] = jnp.dot(z, wl2_ref[...],
                           preferred_element_type=jnp.float32) + bl2_ref[...]


def _tc_call(fn, out_shapes, **kw):
    return pl.pallas_call(fn, out_shape=out_shapes, **kw)


def kernel(x, edge_index, batch, dist, sw, W1, b1, W2, b2, W3, b3,
           Wlin, blin, Wlin1, blin1, Wlin2, blin2):
    f32 = jnp.float32
    src = edge_index[0]
    dst = edge_index[1]
    batchf = jnp.full((NPAD,), 64.0, f32).at[:N].set(batch.astype(f32))
    xpad = jnp.zeros((NPAD,), f32).at[:N].set(x[:, 0])

    a = jnp.maximum(W1[0], 0.0)
    c = jnp.maximum(-W1[0], 0.0)
    alpha = a @ W2
    gamma = c @ W2
    ts = jnp.sort(jnp.where(gamma != 0, -alpha / jnp.where(gamma != 0, gamma, 1.0), 0.0))
    r_rep = jnp.concatenate([ts[:1] - 1.0, (ts[:-1] + ts[1:]) * 0.5, ts[-1:] + 1.0])
    M = ((alpha[None, :] + gamma[None, :] * r_rep[:, None]) > 0.0).astype(f32)

    degp = _sc1_call(dst)

    nshape = jax.ShapeDtypeStruct((SROW, 128), f32)
    dinv2d, gx2d, w2d = _tc_call(_tc1, (nshape, nshape, nshape))(
        degp.reshape(NW, SROW, 128), xpad.reshape(SROW, 128))
    dinv = dinv2d.reshape(NPAD)
    gx = gx2d.reshape(NPAD)

    norm, gxs, paccp = _sc2_call(src, dst, dinv, gx)

    gu2d, gv2d = _tc_call(_tc2, (nshape, nshape))(
        paccp.reshape(NW, SROW, 128), gx2d, dinv2d)
    gu = gu2d.reshape(NPAD)
    gv = gv2d.reshape(NPAD)

    _, _, uaccp, vaccp = _sc3_call(src, dst, gu, gv)

    grid = SROW // 8
    blk_n = pl.BlockSpec((8, 128), lambda i: (i, 0))
    blk_p = pl.BlockSpec((NW, 8, 128), lambda i: (0, i, 0))
    blk_ts = pl.BlockSpec((1, 64), lambda i: (0, 0))
    pu2d, pv2d, t2d = _tc_call(
        _tc3, (nshape, nshape, nshape),
        grid=(grid,),
        in_specs=[blk_p, blk_p, blk_n, blk_n, blk_n, blk_n, blk_ts],
        out_specs=(blk_n, blk_n, blk_n),
    )(uaccp.reshape(NW, SROW, 128), vaccp.reshape(NW, SROW, 128),
      gu2d, gv2d, dinv2d, batchf.reshape(SROW, 128), ts[None, :])
    pu = pu2d.reshape(NPAD)
    pv = pv2d.reshape(NPAD)
    tpk = t2d.reshape(NPAD)

    _, sup, svp, cntp = _sc4_call(src, dst, tpk, pu, pv, norm,
                                  w2d.reshape(NPAD))

    out = _tc_call(_tc4, jax.ShapeDtypeStruct((B, 1), f32))(
        sup[:, :B * NB].reshape(NW, B, NB), svp[:, :B * NB].reshape(NW, B, NB),
        cntp[:, :B], M, alpha[None, :], gamma[None, :], W3, b3[None, :],
        Wlin, blin[None, :], dist, sw, Wlin1, blin1[None, :], Wlin2,
        blin2[None, :])
    return out


# speedup vs baseline: 168.1270x; 1.5115x over previous
import jax
import jax.numpy as jnp
from jax import lax
from jax.experimental import pallas as pl
from jax.experimental.pallas import tpu as pltpu
from jax.experimental.pallas import tpu_sc as plsc

N = 100000
E = 1600000
B = 64
HID = 64

NW = 32
EPW = E // NW
ECH = 2000
NCH = ECH // 16
NCHUNK = EPW // ECH
HALF = (NCHUNK - 1) // 2
NPAD = 100352
NPW = NPAD // NW
NNCH = NPW // 2
NB = 65
SUSZ = 4240
SROW = 784
UNROLL = 8

_mesh = plsc.VectorSubcoreMesh(core_axis_name="c", subcore_axis_name="s")
_sc_params = pltpu.CompilerParams(needs_layout_passes=False)


def _wid():
    return lax.axis_index("c") * 16 + lax.axis_index("s")


def _zero_table(table_v, n):
    z = jnp.zeros((16,), jnp.float32)

    @plsc.parallel_loop(0, n // 16, unroll=UNROLL)
    def _(j):
        table_v[pl.ds(j * 16, 16)] = z


def _pipe(w, ins, outs, compute, si, so):
    def base(i):
        return w * EPW + i * ECH

    def issue_in(i, slot):
        for hbm, bufs in ins:
            pltpu.async_copy(hbm.at[pl.ds(base(i), ECH)], bufs[slot], si[slot])

    def wait_in(slot):
        for hbm, bufs in ins:
            pltpu.make_async_copy(hbm.at[pl.ds(0, ECH)], bufs[slot],
                                  si[slot]).wait()

    def issue_out(i, slot):
        for hbm, obufs in outs:
            pltpu.async_copy(obufs[slot], hbm.at[pl.ds(base(i), ECH)],
                             so[slot])

    def wait_out(slot):
        for hbm, obufs in outs:
            pltpu.make_async_copy(obufs[slot], hbm.at[pl.ds(0, ECH)],
                                  so[slot]).wait()

    issue_in(0, 0)
    issue_in(1, 1)

    def body(c, carry):
        wait_in(0)
        if outs:
            @pl.when(c > 0)
            def _():
                wait_out(0)
        compute(0)
        if outs:
            issue_out(2 * c, 0)

        @pl.when(c < HALF)
        def _():
            issue_in(2 * c + 2, 0)

        wait_in(1)
        if outs:
            @pl.when(c > 0)
            def _():
                wait_out(1)
        compute(1)
        if outs:
            issue_out(2 * c + 1, 1)

        @pl.when(c < HALF - 1)
        def _():
            issue_in(2 * c + 3, 1)

        return carry

    lax.fori_loop(0, HALF, body, None)

    wait_in(0)
    if outs:
        wait_out(0)
    compute(0)
    if outs:
        issue_out(NCHUNK - 1, 0)
        wait_out(0)
        wait_out(1)


def _sc1(dst_hbm, degp_hbm, table_v, i0, i1, si0, si1, so0, so1):
    w = _wid()
    ones = jnp.ones((16,), jnp.float32)
    ib = (i0, i1)

    _zero_table(table_v, NPAD)

    def compute(slot):
        buf = ib[slot]

        @plsc.parallel_loop(0, NCH, unroll=UNROLL)
        def _(j):
            plsc.addupdate_scatter(table_v, [buf[pl.ds(j * 16, 16)]], ones)

    _pipe(w, [(dst_hbm, ib)], [], compute, (si0, si1), (so0, so1))
    pltpu.sync_copy(table_v, degp_hbm.at[w])


_sc1_call = pl.kernel(
    _sc1,
    out_type=jax.ShapeDtypeStruct((NW, NPAD), jnp.float32),
    mesh=_mesh,
    scratch_types=[
        pltpu.VMEM((NPAD,), jnp.float32),
        pltpu.VMEM((ECH,), jnp.int32),
        pltpu.VMEM((ECH,), jnp.int32),
        pltpu.SemaphoreType.DMA,
        pltpu.SemaphoreType.DMA,
        pltpu.SemaphoreType.DMA,
        pltpu.SemaphoreType.DMA,
    ],
    compiler_params=_sc_params,
)


def _sc2(src_hbm, dst_hbm, dinv_hbm, gx_hbm, norm_hbm, gxs_hbm, paccp_hbm,
         table_v, i0, i1, i2, i3, f0, f1, f2, f3, si0, si1, so0, so1):
    w = _wid()
    si = (si0, si1)
    so = (so0, so1)

    pltpu.sync_copy(dinv_hbm, table_v)

    def norm_compute(slot):
        sbuf = (i0, i1)[slot]
        dbuf = (i2, i3)[slot]
        obuf = (f0, f1)[slot]

        @plsc.parallel_loop(0, NCH, unroll=UNROLL)
        def _(j):
            s = sbuf[pl.ds(j * 16, 16)]
            d = dbuf[pl.ds(j * 16, 16)]
            obuf[pl.ds(j * 16, 16)] = (plsc.load_gather(table_v, [s])
                                       * plsc.load_gather(table_v, [d]))

    _pipe(w, [(src_hbm, (i0, i1)), (dst_hbm, (i2, i3))],
          [(norm_hbm, (f0, f1))], norm_compute, si, so)

    pltpu.sync_copy(gx_hbm, table_v)

    def gxs_compute(slot):
        sbuf = (i0, i1)[slot]
        obuf = (f0, f1)[slot]

        @plsc.parallel_loop(0, NCH, unroll=UNROLL)
        def _(j):
            s = sbuf[pl.ds(j * 16, 16)]
            obuf[pl.ds(j * 16, 16)] = plsc.load_gather(table_v, [s])

    _pipe(w, [(src_hbm, (i0, i1))], [(gxs_hbm, (f0, f1))], gxs_compute,
          si, so)

    _zero_table(table_v, NPAD)

    def pacc_compute(slot):
        dbuf = (i0, i1)[slot]
        vbuf = (f2, f3)[slot]

        @plsc.parallel_loop(0, NCH, unroll=UNROLL)
        def _(j):
            d = dbuf[pl.ds(j * 16, 16)]
            plsc.addupdate_scatter(table_v, [d], vbuf[pl.ds(j * 16, 16)])

    _pipe(w, [(dst_hbm, (i0, i1)), (gxs_hbm, (f2, f3))], [], pacc_compute,
          si, so)
    pltpu.sync_copy(table_v, paccp_hbm.at[w])


_sc2_call = pl.kernel(
    _sc2,
    out_type=(
        jax.ShapeDtypeStruct((E,), jnp.float32),
        jax.ShapeDtypeStruct((E,), jnp.float32),
        jax.ShapeDtypeStruct((NW, NPAD), jnp.float32),
    ),
    mesh=_mesh,
    scratch_types=[
        pltpu.VMEM((NPAD,), jnp.float32),
        pltpu.VMEM((ECH,), jnp.int32),
        pltpu.VMEM((ECH,), jnp.int32),
        pltpu.VMEM((ECH,), jnp.int32),
        pltpu.VMEM((ECH,), jnp.int32),
        pltpu.VMEM((ECH,), jnp.float32),
        pltpu.VMEM((ECH,), jnp.float32),
        pltpu.VMEM((ECH,), jnp.float32),
        pltpu.VMEM((ECH,), jnp.float32),
        pltpu.SemaphoreType.DMA,
        pltpu.SemaphoreType.DMA,
        pltpu.SemaphoreType.DMA,
        pltpu.SemaphoreType.DMA,
    ],
    compiler_params=_sc_params,
)


def _sc3(src_hbm, dst_hbm, gu_hbm, gv_hbm, gus_hbm, gvs_hbm,
         uaccp_hbm, vaccp_hbm,
         table_v, i0, i1, f0, f1, f2, f3, si0, si1, so0, so1):
    w = _wid()
    si = (si0, si1)
    so = (so0, so1)

    def gather_phase(tab_hbm, out_hbm):
        pltpu.sync_copy(tab_hbm, table_v)

        def compute(slot):
            sbuf = (i0, i1)[slot]
            obuf = (f0, f1)[slot]

            @plsc.parallel_loop(0, NCH, unroll=UNROLL)
            def _(j):
                s = sbuf[pl.ds(j * 16, 16)]
                obuf[pl.ds(j * 16, 16)] = plsc.load_gather(table_v, [s])

        _pipe(w, [(src_hbm, (i0, i1))], [(out_hbm, (f0, f1))], compute,
              si, so)

    def scatter_phase(vals_hbm, out_hbm):
        _zero_table(table_v, NPAD)

        def compute(slot):
            dbuf = (i0, i1)[slot]
            vbuf = (f2, f3)[slot]

            @plsc.parallel_loop(0, NCH, unroll=UNROLL)
            def _(j):
                d = dbuf[pl.ds(j * 16, 16)]
                plsc.addupdate_scatter(table_v, [d], vbuf[pl.ds(j * 16, 16)])

        _pipe(w, [(dst_hbm, (i0, i1)), (vals_hbm, (f2, f3))], [], compute,
              si, so)
        pltpu.sync_copy(table_v, out_hbm.at[w])

    gather_phase(gu_hbm, gus_hbm)
    gather_phase(gv_hbm, gvs_hbm)
    scatter_phase(gus_hbm, uaccp_hbm)
    scatter_phase(gvs_hbm, vaccp_hbm)


_sc3_call = pl.kernel(
    _sc3,
    out_type=(
        jax.ShapeDtypeStruct((E,), jnp.float32),
        jax.ShapeDtypeStruct((E,), jnp.float32),
        jax.ShapeDtypeStruct((NW, NPAD), jnp.float32),
        jax.ShapeDtypeStruct((NW, NPAD), jnp.float32),
    ),
    mesh=_mesh,
    scratch_types=[
        pltpu.VMEM((NPAD,), jnp.float32),
        pltpu.VMEM((ECH,), jnp.int32),
        pltpu.VMEM((ECH,), jnp.int32),
        pltpu.VMEM((ECH,), jnp.float32),
        pltpu.VMEM((ECH,), jnp.float32),
        pltpu.VMEM((ECH,), jnp.float32),
        pltpu.VMEM((ECH,), jnp.float32),
        pltpu.SemaphoreType.DMA,
        pltpu.SemaphoreType.DMA,
        pltpu.SemaphoreType.DMA,
        pltpu.SemaphoreType.DMA,
    ],
    compiler_params=_sc_params,
)


def _sc4(src_hbm, dst_hbm, t_hbm, pu_hbm, pv_hbm, norm_hbm, wslf_hbm,
         flat_hbm, sup_hbm, svp_hbm, cntp_hbm,
         table_v, i0, i1, i2, i3, f0, f1, f2, f3, f4, f5, su_v, sv_v, cnt_v,
         si0, si1, so0, so1):
    w = _wid()
    si = (si0, si1)
    so = (so0, so1)

    pltpu.sync_copy(t_hbm, table_v)

    def flat_compute(slot):
        sbuf = (i0, i1)[slot]
        dbuf = (i2, i3)[slot]
        obuf = (f0, f1)[slot]

        @plsc.parallel_loop(0, NCH, unroll=UNROLL)
        def _(j):
            s = sbuf[pl.ds(j * 16, 16)]
            d = dbuf[pl.ds(j * 16, 16)]
            ts_ = plsc.load_gather(table_v, [s]).astype(jnp.int32)
            td = plsc.load_gather(table_v, [d]).astype(jnp.int32)
            flat = (td >> 8) * NB + (ts_ & 255)
            obuf[pl.ds(j * 16, 16)] = flat.astype(jnp.float32)

    _pipe(w, [(src_hbm, (i0, i1)), (dst_hbm, (i2, i3))],
          [(flat_hbm, (f0, f1))], flat_compute, si, so)

    _zero_table(su_v, SUSZ)
    _zero_table(sv_v, SUSZ)
    _zero_table(cnt_v, 80)

    def fold_phase(tab_hbm, acc_v):
        pltpu.sync_copy(tab_hbm, table_v)

        def compute(slot):
            sbuf = (i0, i1)[slot]
            flbuf = (f0, f1)[slot]
            nmbuf = (f2, f3)[slot]

            @plsc.parallel_loop(0, NCH, unroll=UNROLL)
            def _(j):
                s = sbuf[pl.ds(j * 16, 16)]
                g = plsc.load_gather(table_v, [s])
                fl = flbuf[pl.ds(j * 16, 16)].astype(jnp.int32)
                nm = nmbuf[pl.ds(j * 16, 16)]
                plsc.addupdate_scatter(acc_v, [fl], nm * g)

        _pipe(w, [(src_hbm, (i0, i1)), (flat_hbm, (f0, f1)),
                  (norm_hbm, (f2, f3))], [], compute, si, so)

    fold_phase(pu_hbm, su_v)
    fold_phase(pv_hbm, sv_v)

    ones = jnp.ones((16,), jnp.float32)

    def node_chunk(c, carry):
        base = w * NPW + c * NNCH
        pltpu.sync_copy(t_hbm.at[pl.ds(base, NNCH)], f0.at[pl.ds(0, NNCH)])
        pltpu.sync_copy(pu_hbm.at[pl.ds(base, NNCH)], f2.at[pl.ds(0, NNCH)])
        pltpu.sync_copy(pv_hbm.at[pl.ds(base, NNCH)], f3.at[pl.ds(0, NNCH)])
        pltpu.sync_copy(wslf_hbm.at[pl.ds(base, NNCH)], f4.at[pl.ds(0, NNCH)])

        @plsc.parallel_loop(0, NNCH // 16, unroll=UNROLL)
        def _(j):
            t = f0[pl.ds(j * 16, 16)].astype(jnp.int32)
            sg = t >> 8
            fl = sg * NB + (t & 255)
            wt = f4[pl.ds(j * 16, 16)]
            plsc.addupdate_scatter(su_v, [fl], wt * f2[pl.ds(j * 16, 16)])
            plsc.addupdate_scatter(sv_v, [fl], wt * f3[pl.ds(j * 16, 16)])
            plsc.addupdate_scatter(cnt_v, [sg], ones)

        return carry

    lax.fori_loop(0, 2, node_chunk, None)

    pltpu.sync_copy(su_v, sup_hbm.at[w])
    pltpu.sync_copy(sv_v, svp_hbm.at[w])
    pltpu.sync_copy(cnt_v, cntp_hbm.at[w])


_sc4_call = pl.kernel(
    _sc4,
    out_type=(
        jax.ShapeDtypeStruct((E,), jnp.float32),
        jax.ShapeDtypeStruct((NW, SUSZ), jnp.float32),
        jax.ShapeDtypeStruct((NW, SUSZ), jnp.float32),
        jax.ShapeDtypeStruct((NW, 80), jnp.float32),
    ),
    mesh=_mesh,
    scratch_types=[
        pltpu.VMEM((NPAD,), jnp.float32),
        pltpu.VMEM((ECH,), jnp.int32),
        pltpu.VMEM((ECH,), jnp.int32),
        pltpu.VMEM((ECH,), jnp.int32),
        pltpu.VMEM((ECH,), jnp.int32),
        pltpu.VMEM((ECH,), jnp.float32),
        pltpu.VMEM((ECH,), jnp.float32),
        pltpu.VMEM((ECH,), jnp.float32),
        pltpu.VMEM((ECH,), jnp.float32),
        pltpu.VMEM((ECH,), jnp.float32),
        pltpu.VMEM((ECH,), jnp.float32),
        pltpu.VMEM((SUSZ,), jnp.float32),
        pltpu.VMEM((SUSZ,), jnp.float32),
        pltpu.VMEM((80,), jnp.float32),
        pltpu.SemaphoreType.DMA,
        pltpu.SemaphoreType.DMA,
        pltpu.SemaphoreType.DMA,
        pltpu.SemaphoreType.DMA,
    ],
    compiler_params=_sc_params,
)


def _tc1(degp_ref, x_ref, dinv_ref, gx_ref, w_ref):
    deg = jnp.sum(degp_ref[...], axis=0) + 1.0
    dinv = lax.rsqrt(deg)
    dinv_ref[...] = dinv
    gx_ref[...] = dinv * x_ref[...]
    w_ref[...] = dinv * dinv


def _tc2(paccp_ref, gx_ref, dinv_ref, gu_ref, gv_ref):
    acc = jnp.sum(paccp_ref[...], axis=0)
    dinv = dinv_ref[...]
    p = dinv * (acc + gx_ref[...])
    gu_ref[...] = dinv * jnp.maximum(p, 0.0)
    gv_ref[...] = dinv * jnp.maximum(-p, 0.0)


def _tc3(uaccp_ref, vaccp_ref, gu_ref, gv_ref, dinv_ref, batchf_ref, ts_ref,
         pu_ref, pv_ref, t_ref):
    dinv = dinv_ref[...]
    pu = dinv * (jnp.sum(uaccp_ref[...], axis=0) + gu_ref[...])
    pv = dinv * (jnp.sum(vaccp_ref[...], axis=0) + gv_ref[...])
    pu_ref[...] = pu
    pv_ref[...] = pv
    r = pv / pu
    bk = jnp.sum((ts_ref[...][0][None, None, :] < r[:, :, None]).astype(jnp.float32),
                 axis=-1)
    t_ref[...] = batchf_ref[...] * 256.0 + bk


def _tc4(sup_ref, svp_ref, cntp_ref, m_ref, al_ref, ga_ref, w3_ref, b3_ref,
         wl_ref, bl_ref, ds_ref, sw_ref, wl1_ref, bl1_ref, wl2_ref, bl2_ref,
         out_ref):
    su = jnp.sum(sup_ref[...], axis=0)
    sv = jnp.sum(svp_ref[...], axis=0)
    cnt = jnp.maximum(jnp.sum(cntp_ref[...], axis=0), 1.0)
    m = m_ref[...]
    a_mat = (al_ref[...] * jnp.dot(su, m, preferred_element_type=jnp.float32)
             + ga_ref[...] * jnp.dot(sv, m, preferred_element_type=jnp.float32))
    pooled = jnp.dot(a_mat / cnt[:, None], w3_ref[...],
                     preferred_element_type=jnp.float32) + b3_ref[...]
    z = jnp.dot(pooled, wl_ref[...],
                preferred_element_type=jnp.float32) + bl_ref[...]
    z = jnp.concatenate([z, ds_ref[...], sw_ref[...]], axis=1)
    z = jnp.maximum(jnp.dot(z, wl1_ref[...],
                            preferred_element_type=jnp.float32) + bl1_ref[...], 0.0)
    out_ref[...] = jnp.dot(z, wl2_ref[...],
                           preferred_element_type=jnp.float32) + bl2_ref[...]


def _tc_call(fn, out_shapes, **kw):
    return pl.pallas_call(fn, out_shape=out_shapes, **kw)


def kernel(x, edge_index, batch, dist, sw, W1, b1, W2, b2, W3, b3,
           Wlin, blin, Wlin1, blin1, Wlin2, blin2):
    f32 = jnp.float32
    src = edge_index[0]
    dst = edge_index[1]
    batchf = jnp.full((NPAD,), 64.0, f32).at[:N].set(batch.astype(f32))
    xpad = jnp.zeros((NPAD,), f32).at[:N].set(x[:, 0])

    a = jnp.maximum(W1[0], 0.0)
    c = jnp.maximum(-W1[0], 0.0)
    alpha = a @ W2
    gamma = c @ W2
    ts = jnp.sort(jnp.where(gamma != 0, -alpha / jnp.where(gamma != 0, gamma, 1.0), 0.0))
    r_rep = jnp.concatenate([ts[:1] - 1.0, (ts[:-1] + ts[1:]) * 0.5, ts[-1:] + 1.0])
    M = ((alpha[None, :] + gamma[None, :] * r_rep[:, None]) > 0.0).astype(f32)

    degp = _sc1_call(dst)

    nshape = jax.ShapeDtypeStruct((SROW, 128), f32)
    dinv2d, gx2d, w2d = _tc_call(_tc1, (nshape, nshape, nshape))(
        degp.reshape(NW, SROW, 128), xpad.reshape(SROW, 128))
    dinv = dinv2d.reshape(NPAD)
    gx = gx2d.reshape(NPAD)

    norm, gxs, paccp = _sc2_call(src, dst, dinv, gx)

    gu2d, gv2d = _tc_call(_tc2, (nshape, nshape))(
        paccp.reshape(NW, SROW, 128), gx2d, dinv2d)
    gu = gu2d.reshape(NPAD)
    gv = gv2d.reshape(NPAD)

    _, _, uaccp, vaccp = _sc3_call(src, dst, gu, gv)

    grid = SROW // 8
    blk_n = pl.BlockSpec((8, 128), lambda i: (i, 0))
    blk_p = pl.BlockSpec((NW, 8, 128), lambda i: (0, i, 0))
    blk_ts = pl.BlockSpec((1, 64), lambda i: (0, 0))
    pu2d, pv2d, t2d = _tc_call(
        _tc3, (nshape, nshape, nshape),
        grid=(grid,),
        in_specs=[blk_p, blk_p, blk_n, blk_n, blk_n, blk_n, blk_ts],
        out_specs=(blk_n, blk_n, blk_n),
    )(uaccp.reshape(NW, SROW, 128), vaccp.reshape(NW, SROW, 128),
      gu2d, gv2d, dinv2d, batchf.reshape(SROW, 128), ts[None, :])
    pu = pu2d.reshape(NPAD)
    pv = pv2d.reshape(NPAD)
    tpk = t2d.reshape(NPAD)

    _, sup, svp, cntp = _sc4_call(src, dst, tpk, pu, pv, norm,
                                  w2d.reshape(NPAD))

    out = _tc_call(_tc4, jax.ShapeDtypeStruct((B, 1), f32))(
        sup[:, :B * NB].reshape(NW, B, NB), svp[:, :B * NB].reshape(NW, B, NB),
        cntp[:, :B], M, alpha[None, :], gamma[None, :], W3, b3[None, :],
        Wlin, blin[None, :], dist, sw, Wlin1, blin1[None, :], Wlin2,
        blin2[None, :])
    return out


# speedup vs baseline: 184.7595x; 1.0989x over previous
import jax
import jax.numpy as jnp
from jax import lax
from jax.experimental import pallas as pl
from jax.experimental.pallas import tpu as pltpu
from jax.experimental.pallas import tpu_sc as plsc

N = 100000
E = 1600000
B = 64
HID = 64

NW = 32
EPW = E // NW
ECH = 2000
NCH = ECH // 16
NCHUNK = EPW // ECH
HALF = (NCHUNK - 1) // 2
NPAD = 100352
NPW = NPAD // NW
NNCH = NPW // 2
NB = 65
SUSZ = 4240
SROW = 784
UNROLL = 8

_mesh = plsc.VectorSubcoreMesh(core_axis_name="c", subcore_axis_name="s")
_sc_params = pltpu.CompilerParams(needs_layout_passes=False)


def _wid():
    return lax.axis_index("c") * 16 + lax.axis_index("s")


def _zero_table(table_v, n):
    z = jnp.zeros((16,), jnp.float32)

    @plsc.parallel_loop(0, n // 16, unroll=UNROLL)
    def _(j):
        table_v[pl.ds(j * 16, 16)] = z


def _pipe(w, ins, outs, compute, si, so):
    def base(i):
        return w * EPW + i * ECH

    def issue_in(i, slot):
        for hbm, bufs in ins:
            pltpu.async_copy(hbm.at[pl.ds(base(i), ECH)], bufs[slot], si[slot])

    def wait_in(slot):
        for hbm, bufs in ins:
            pltpu.make_async_copy(hbm.at[pl.ds(0, ECH)], bufs[slot],
                                  si[slot]).wait()

    def issue_out(i, slot):
        for hbm, obufs in outs:
            pltpu.async_copy(obufs[slot], hbm.at[pl.ds(base(i), ECH)],
                             so[slot])

    def wait_out(slot):
        for hbm, obufs in outs:
            pltpu.make_async_copy(obufs[slot], hbm.at[pl.ds(0, ECH)],
                                  so[slot]).wait()

    issue_in(0, 0)
    issue_in(1, 1)

    def body(c, carry):
        wait_in(0)
        if outs:
            @pl.when(c > 0)
            def _():
                wait_out(0)
        compute(0)
        if outs:
            issue_out(2 * c, 0)

        @pl.when(c < HALF)
        def _():
            issue_in(2 * c + 2, 0)

        wait_in(1)
        if outs:
            @pl.when(c > 0)
            def _():
                wait_out(1)
        compute(1)
        if outs:
            issue_out(2 * c + 1, 1)

        @pl.when(c < HALF - 1)
        def _():
            issue_in(2 * c + 3, 1)

        return carry

    lax.fori_loop(0, HALF, body, None)

    wait_in(0)
    if outs:
        wait_out(0)
    compute(0)
    if outs:
        issue_out(NCHUNK - 1, 0)
        wait_out(0)
        wait_out(1)


def _sc1(dst_hbm, degp_hbm, table_v, i0, i1, si0, si1, so0, so1):
    w = _wid()
    ones = jnp.ones((16,), jnp.float32)
    ib = (i0, i1)

    _zero_table(table_v, NPAD)

    def compute(slot):
        buf = ib[slot]

        @plsc.parallel_loop(0, NCH, unroll=UNROLL)
        def _(j):
            plsc.addupdate_scatter(table_v, [buf[pl.ds(j * 16, 16)]], ones)

    _pipe(w, [(dst_hbm, ib)], [], compute, (si0, si1), (so0, so1))
    pltpu.sync_copy(table_v, degp_hbm.at[w])


_sc1_call = pl.kernel(
    _sc1,
    out_type=jax.ShapeDtypeStruct((NW, NPAD), jnp.float32),
    mesh=_mesh,
    scratch_types=[
        pltpu.VMEM((NPAD,), jnp.float32),
        pltpu.VMEM((ECH,), jnp.int32),
        pltpu.VMEM((ECH,), jnp.int32),
        pltpu.SemaphoreType.DMA,
        pltpu.SemaphoreType.DMA,
        pltpu.SemaphoreType.DMA,
        pltpu.SemaphoreType.DMA,
    ],
    compiler_params=_sc_params,
)


def _sc2(src_hbm, dst_hbm, dinv_hbm, gx_hbm, norm_hbm, gxs_hbm, paccp_hbm,
         table_v, i0, i1, i2, i3, f0, f1, f2, f3, si0, si1, so0, so1):
    w = _wid()
    si = (si0, si1)
    so = (so0, so1)

    pltpu.sync_copy(dinv_hbm, table_v)

    def norm_compute(slot):
        sbuf = (i0, i1)[slot]
        dbuf = (i2, i3)[slot]
        obuf = (f0, f1)[slot]

        @plsc.parallel_loop(0, NCH, unroll=UNROLL)
        def _(j):
            s = sbuf[pl.ds(j * 16, 16)]
            d = dbuf[pl.ds(j * 16, 16)]
            obuf[pl.ds(j * 16, 16)] = (plsc.load_gather(table_v, [s])
                                       * plsc.load_gather(table_v, [d]))

    _pipe(w, [(src_hbm, (i0, i1)), (dst_hbm, (i2, i3))],
          [(norm_hbm, (f0, f1))], norm_compute, si, so)

    pltpu.sync_copy(gx_hbm, table_v)

    def gxs_compute(slot):
        sbuf = (i0, i1)[slot]
        obuf = (f0, f1)[slot]

        @plsc.parallel_loop(0, NCH, unroll=UNROLL)
        def _(j):
            s = sbuf[pl.ds(j * 16, 16)]
            obuf[pl.ds(j * 16, 16)] = plsc.load_gather(table_v, [s])

    _pipe(w, [(src_hbm, (i0, i1))], [(gxs_hbm, (f0, f1))], gxs_compute,
          si, so)

    _zero_table(table_v, NPAD)

    def pacc_compute(slot):
        dbuf = (i0, i1)[slot]
        vbuf = (f2, f3)[slot]

        @plsc.parallel_loop(0, NCH, unroll=UNROLL)
        def _(j):
            d = dbuf[pl.ds(j * 16, 16)]
            plsc.addupdate_scatter(table_v, [d], vbuf[pl.ds(j * 16, 16)])

    _pipe(w, [(dst_hbm, (i0, i1)), (gxs_hbm, (f2, f3))], [], pacc_compute,
          si, so)
    pltpu.sync_copy(table_v, paccp_hbm.at[w])


_sc2_call = pl.kernel(
    _sc2,
    out_type=(
        jax.ShapeDtypeStruct((E,), jnp.float32),
        jax.ShapeDtypeStruct((E,), jnp.float32),
        jax.ShapeDtypeStruct((NW, NPAD), jnp.float32),
    ),
    mesh=_mesh,
    scratch_types=[
        pltpu.VMEM((NPAD,), jnp.float32),
        pltpu.VMEM((ECH,), jnp.int32),
        pltpu.VMEM((ECH,), jnp.int32),
        pltpu.VMEM((ECH,), jnp.int32),
        pltpu.VMEM((ECH,), jnp.int32),
        pltpu.VMEM((ECH,), jnp.float32),
        pltpu.VMEM((ECH,), jnp.float32),
        pltpu.VMEM((ECH,), jnp.float32),
        pltpu.VMEM((ECH,), jnp.float32),
        pltpu.SemaphoreType.DMA,
        pltpu.SemaphoreType.DMA,
        pltpu.SemaphoreType.DMA,
        pltpu.SemaphoreType.DMA,
    ],
    compiler_params=_sc_params,
)


def _sc3(src_hbm, dst_hbm, gs_hbm, gss_hbm, uaccp_hbm, vaccp_hbm,
         table_v, i0, i1, f0, f1, f2, f3, si0, si1, so0, so1):
    w = _wid()
    si = (si0, si1)
    so = (so0, so1)

    pltpu.sync_copy(gs_hbm, table_v)

    def gather_compute(slot):
        sbuf = (i0, i1)[slot]
        obuf = (f0, f1)[slot]

        @plsc.parallel_loop(0, NCH, unroll=UNROLL)
        def _(j):
            s = sbuf[pl.ds(j * 16, 16)]
            obuf[pl.ds(j * 16, 16)] = plsc.load_gather(table_v, [s])

    _pipe(w, [(src_hbm, (i0, i1))], [(gss_hbm, (f0, f1))], gather_compute,
          si, so)

    def scatter_phase(sign, out_hbm):
        _zero_table(table_v, NPAD)

        def compute(slot):
            dbuf = (i0, i1)[slot]
            vbuf = (f2, f3)[slot]

            @plsc.parallel_loop(0, NCH, unroll=UNROLL)
            def _(j):
                d = dbuf[pl.ds(j * 16, 16)]
                v = jnp.maximum(sign * vbuf[pl.ds(j * 16, 16)], 0.0)
                plsc.addupdate_scatter(table_v, [d], v)

        _pipe(w, [(dst_hbm, (i0, i1)), (gss_hbm, (f2, f3))], [], compute,
              si, so)
        pltpu.sync_copy(table_v, out_hbm.at[w])

    scatter_phase(1.0, uaccp_hbm)
    scatter_phase(-1.0, vaccp_hbm)


_sc3_call = pl.kernel(
    _sc3,
    out_type=(
        jax.ShapeDtypeStruct((E,), jnp.float32),
        jax.ShapeDtypeStruct((NW, NPAD), jnp.float32),
        jax.ShapeDtypeStruct((NW, NPAD), jnp.float32),
    ),
    mesh=_mesh,
    scratch_types=[
        pltpu.VMEM((NPAD,), jnp.float32),
        pltpu.VMEM((ECH,), jnp.int32),
        pltpu.VMEM((ECH,), jnp.int32),
        pltpu.VMEM((ECH,), jnp.float32),
        pltpu.VMEM((ECH,), jnp.float32),
        pltpu.VMEM((ECH,), jnp.float32),
        pltpu.VMEM((ECH,), jnp.float32),
        pltpu.SemaphoreType.DMA,
        pltpu.SemaphoreType.DMA,
        pltpu.SemaphoreType.DMA,
        pltpu.SemaphoreType.DMA,
    ],
    compiler_params=_sc_params,
)


def _sc4(src_hbm, dst_hbm, t_hbm, pk_hbm, pu_hbm, pv_hbm, norm_hbm, wslf_hbm,
         flat_hbm, sup_hbm, svp_hbm, cntp_hbm,
         table_v, i0, i1, i2, i3, f0, f1, f2, f3, f4, f5, su_v, sv_v, cnt_v,
         si0, si1, so0, so1):
    w = _wid()
    si = (si0, si1)
    so = (so0, so1)

    pltpu.sync_copy(t_hbm, table_v)

    def flat_compute(slot):
        sbuf = (i0, i1)[slot]
        dbuf = (i2, i3)[slot]
        obuf = (f0, f1)[slot]

        @plsc.parallel_loop(0, NCH, unroll=UNROLL)
        def _(j):
            s = sbuf[pl.ds(j * 16, 16)]
            d = dbuf[pl.ds(j * 16, 16)]
            ts_ = plsc.load_gather(table_v, [s]).astype(jnp.int32)
            td = plsc.load_gather(table_v, [d]).astype(jnp.int32)
            flat = (td >> 8) * NB + (ts_ & 255)
            obuf[pl.ds(j * 16, 16)] = flat.astype(jnp.float32)

    _pipe(w, [(src_hbm, (i0, i1)), (dst_hbm, (i2, i3))],
          [(flat_hbm, (f0, f1))], flat_compute, si, so)

    _zero_table(su_v, SUSZ)
    _zero_table(sv_v, SUSZ)
    _zero_table(cnt_v, 80)

    pltpu.sync_copy(pk_hbm, table_v)
    himask = jnp.full((16,), -65536, jnp.int32)

    def fold_compute(slot):
        sbuf = (i0, i1)[slot]
        flbuf = (f0, f1)[slot]
        nmbuf = (f2, f3)[slot]

        @plsc.parallel_loop(0, NCH, unroll=UNROLL)
        def _(j):
            s = sbuf[pl.ds(j * 16, 16)]
            g = plsc.bitcast(plsc.load_gather(table_v, [s]), jnp.int32)
            pu_s = plsc.bitcast(g & himask, jnp.float32)
            pv_s = plsc.bitcast(g << 16, jnp.float32)
            fl = flbuf[pl.ds(j * 16, 16)].astype(jnp.int32)
            nm = nmbuf[pl.ds(j * 16, 16)]
            plsc.addupdate_scatter(su_v, [fl], nm * pu_s)
            plsc.addupdate_scatter(sv_v, [fl], nm * pv_s)

    _pipe(w, [(src_hbm, (i0, i1)), (flat_hbm, (f0, f1)),
              (norm_hbm, (f2, f3))], [], fold_compute, si, so)

    ones = jnp.ones((16,), jnp.float32)

    def node_chunk(c, carry):
        base = w * NPW + c * NNCH
        pltpu.sync_copy(t_hbm.at[pl.ds(base, NNCH)], f0.at[pl.ds(0, NNCH)])
        pltpu.sync_copy(pu_hbm.at[pl.ds(base, NNCH)], f2.at[pl.ds(0, NNCH)])
        pltpu.sync_copy(pv_hbm.at[pl.ds(base, NNCH)], f3.at[pl.ds(0, NNCH)])
        pltpu.sync_copy(wslf_hbm.at[pl.ds(base, NNCH)], f4.at[pl.ds(0, NNCH)])

        @plsc.parallel_loop(0, NNCH // 16, unroll=UNROLL)
        def _(j):
            t = f0[pl.ds(j * 16, 16)].astype(jnp.int32)
            sg = t >> 8
            fl = sg * NB + (t & 255)
            wt = f4[pl.ds(j * 16, 16)]
            plsc.addupdate_scatter(su_v, [fl], wt * f2[pl.ds(j * 16, 16)])
            plsc.addupdate_scatter(sv_v, [fl], wt * f3[pl.ds(j * 16, 16)])
            plsc.addupdate_scatter(cnt_v, [sg], ones)

        return carry

    lax.fori_loop(0, 2, node_chunk, None)

    pltpu.sync_copy(su_v, sup_hbm.at[w])
    pltpu.sync_copy(sv_v, svp_hbm.at[w])
    pltpu.sync_copy(cnt_v, cntp_hbm.at[w])


_sc4_call = pl.kernel(
    _sc4,
    out_type=(
        jax.ShapeDtypeStruct((E,), jnp.float32),
        jax.ShapeDtypeStruct((NW, SUSZ), jnp.float32),
        jax.ShapeDtypeStruct((NW, SUSZ), jnp.float32),
        jax.ShapeDtypeStruct((NW, 80), jnp.float32),
    ),
    mesh=_mesh,
    scratch_types=[
        pltpu.VMEM((NPAD,), jnp.float32),
        pltpu.VMEM((ECH,), jnp.int32),
        pltpu.VMEM((ECH,), jnp.int32),
        pltpu.VMEM((ECH,), jnp.int32),
        pltpu.VMEM((ECH,), jnp.int32),
        pltpu.VMEM((ECH,), jnp.float32),
        pltpu.VMEM((ECH,), jnp.float32),
        pltpu.VMEM((ECH,), jnp.float32),
        pltpu.VMEM((ECH,), jnp.float32),
        pltpu.VMEM((ECH,), jnp.float32),
        pltpu.VMEM((ECH,), jnp.float32),
        pltpu.VMEM((SUSZ,), jnp.float32),
        pltpu.VMEM((SUSZ,), jnp.float32),
        pltpu.VMEM((80,), jnp.float32),
        pltpu.SemaphoreType.DMA,
        pltpu.SemaphoreType.DMA,
        pltpu.SemaphoreType.DMA,
        pltpu.SemaphoreType.DMA,
    ],
    compiler_params=_sc_params,
)


def _tc1(degp_ref, x_ref, dinv_ref, gx_ref, w_ref):
    deg = jnp.sum(degp_ref[...], axis=0) + 1.0
    dinv = lax.rsqrt(deg)
    dinv_ref[...] = dinv
    gx_ref[...] = dinv * x_ref[...]
    w_ref[...] = dinv * dinv


def _tc2(paccp_ref, gx_ref, dinv_ref, gs_ref):
    acc = jnp.sum(paccp_ref[...], axis=0)
    dinv = dinv_ref[...]
    p = dinv * (acc + gx_ref[...])
    gs_ref[...] = dinv * p


def _tc3(uaccp_ref, vaccp_ref, gs_ref, dinv_ref, batchf_ref, ts_ref,
         pu_ref, pv_ref, t_ref, pk_ref):
    dinv = dinv_ref[...]
    gs = gs_ref[...]
    pu = dinv * (jnp.sum(uaccp_ref[...], axis=0) + jnp.maximum(gs, 0.0))
    pv = dinv * (jnp.sum(vaccp_ref[...], axis=0) + jnp.maximum(-gs, 0.0))
    pu_ref[...] = pu
    pv_ref[...] = pv
    r = pv / pu
    bk = jnp.sum((ts_ref[...][0][None, None, :] < r[:, :, None]).astype(jnp.float32),
                 axis=-1)
    t_ref[...] = batchf_ref[...] * 256.0 + bk
    au = lax.bitcast_convert_type(pu, jnp.int32)
    av = lax.bitcast_convert_type(pv, jnp.int32)
    pk = (au & jnp.int32(-65536)) | lax.shift_right_logical(av, 16)
    pk_ref[...] = lax.bitcast_convert_type(pk, jnp.float32)


def _tc4(sup_ref, svp_ref, cntp_ref, m_ref, al_ref, ga_ref, w3_ref, b3_ref,
         wl_ref, bl_ref, ds_ref, sw_ref, wl1_ref, bl1_ref, wl2_ref, bl2_ref,
         out_ref):
    su = jnp.sum(sup_ref[...], axis=0)
    sv = jnp.sum(svp_ref[...], axis=0)
    cnt = jnp.maximum(jnp.sum(cntp_ref[...], axis=0), 1.0)
    m = m_ref[...]
    a_mat = (al_ref[...] * jnp.dot(su, m, preferred_element_type=jnp.float32)
             + ga_ref[...] * jnp.dot(sv, m, preferred_element_type=jnp.float32))
    pooled = jnp.dot(a_mat / cnt[:, None], w3_ref[...],
                     preferred_element_type=jnp.float32) + b3_ref[...]
    z = jnp.dot(pooled, wl_ref[...],
                preferred_element_type=jnp.float32) + bl_ref[...]
    z = jnp.concatenate([z, ds_ref[...], sw_ref[...]], axis=1)
    z = jnp.maximum(jnp.dot(z, wl1_ref[...],
                            preferred_element_type=jnp.float32) + bl1_ref[...], 0.0)
    out_ref[...] = jnp.dot(z, wl2_ref[...],
                           preferred_element_type=jnp.float32) + bl2_ref[...]


def _tc_call(fn, out_shapes, **kw):
    return pl.pallas_call(fn, out_shape=out_shapes, **kw)


def kernel(x, edge_index, batch, dist, sw, W1, b1, W2, b2, W3, b3,
           Wlin, blin, Wlin1, blin1, Wlin2, blin2):
    f32 = jnp.float32
    src = edge_index[0]
    dst = edge_index[1]
    batchf = jnp.full((NPAD,), 64.0, f32).at[:N].set(batch.astype(f32))
    xpad = jnp.zeros((NPAD,), f32).at[:N].set(x[:, 0])

    a = jnp.maximum(W1[0], 0.0)
    c = jnp.maximum(-W1[0], 0.0)
    alpha = a @ W2
    gamma = c @ W2
    ts = jnp.sort(jnp.where(gamma != 0, -alpha / jnp.where(gamma != 0, gamma, 1.0), 0.0))
    r_rep = jnp.concatenate([ts[:1] - 1.0, (ts[:-1] + ts[1:]) * 0.5, ts[-1:] + 1.0])
    M = ((alpha[None, :] + gamma[None, :] * r_rep[:, None]) > 0.0).astype(f32)

    degp = _sc1_call(dst)

    nshape = jax.ShapeDtypeStruct((SROW, 128), f32)
    dinv2d, gx2d, w2d = _tc_call(_tc1, (nshape, nshape, nshape))(
        degp.reshape(NW, SROW, 128), xpad.reshape(SROW, 128))
    dinv = dinv2d.reshape(NPAD)
    gx = gx2d.reshape(NPAD)

    norm, gxs, paccp = _sc2_call(src, dst, dinv, gx)

    gs2d = _tc_call(_tc2, nshape)(paccp.reshape(NW, SROW, 128), gx2d, dinv2d)
    gs = gs2d.reshape(NPAD)

    _, uaccp, vaccp = _sc3_call(src, dst, gs)

    grid = SROW // 8
    blk_n = pl.BlockSpec((8, 128), lambda i: (i, 0))
    blk_p = pl.BlockSpec((NW, 8, 128), lambda i: (0, i, 0))
    blk_ts = pl.BlockSpec((1, 64), lambda i: (0, 0))
    pu2d, pv2d, t2d, pk2d = _tc_call(
        _tc3, (nshape, nshape, nshape, nshape),
        grid=(grid,),
        in_specs=[blk_p, blk_p, blk_n, blk_n, blk_n, blk_ts],
        out_specs=(blk_n, blk_n, blk_n, blk_n),
    )(uaccp.reshape(NW, SROW, 128), vaccp.reshape(NW, SROW, 128),
      gs2d, dinv2d, batchf.reshape(SROW, 128), ts[None, :])
    pu = pu2d.reshape(NPAD)
    pv = pv2d.reshape(NPAD)
    tpk = t2d.reshape(NPAD)

    _, sup, svp, cntp = _sc4_call(src, dst, tpk, pk2d.reshape(NPAD), pu, pv,
                                  norm, w2d.reshape(NPAD))

    out = _tc_call(_tc4, jax.ShapeDtypeStruct((B, 1), f32))(
        sup[:, :B * NB].reshape(NW, B, NB), svp[:, :B * NB].reshape(NW, B, NB),
        cntp[:, :B], M, alpha[None, :], gamma[None, :], W3, b3[None, :],
        Wlin, blin[None, :], dist, sw, Wlin1, blin1[None, :], Wlin2,
        blin2[None, :])
    return out


# speedup vs baseline: 208.5244x; 1.1286x over previous
import jax
import jax.numpy as jnp
from jax import lax
from jax.experimental import pallas as pl
from jax.experimental.pallas import tpu as pltpu
from jax.experimental.pallas import tpu_sc as plsc

N = 100000
E = 1600000
B = 64
HID = 64

NW = 32
EPW = E // NW
ECH = 2000
NCH = ECH // 16
NCHUNK = EPW // ECH
HALF = (NCHUNK - 1) // 2
NPAD = 100352
NPW = NPAD // NW
NNCH = NPW // 2
NB = 65
SUSZ = 4240
SROW = 784
UNROLL = 8

_mesh = plsc.VectorSubcoreMesh(core_axis_name="c", subcore_axis_name="s")
_sc_params = pltpu.CompilerParams(needs_layout_passes=False)


def _wid():
    return lax.axis_index("c") * 16 + lax.axis_index("s")


def _zero_table(table_v, n):
    z = jnp.zeros((16,), jnp.float32)

    @plsc.parallel_loop(0, n // 16, unroll=UNROLL)
    def _(j):
        table_v[pl.ds(j * 16, 16)] = z


def _zero_table2d(table_v):
    z = jnp.zeros((16,), jnp.float32)

    @plsc.parallel_loop(0, SROW * 8, unroll=UNROLL)
    def _(j):
        table_v[j >> 3, pl.ds((j & 7) * 16, 16)] = z


def _gat(table_v, idx):
    return plsc.load_gather(table_v, [idx >> 7, idx & 127])


def _sca(table_v, idx, val):
    plsc.addupdate_scatter(table_v, [idx >> 7, idx & 127], val)


def _pipe(w, ins, outs, compute, si, so):
    def base(i):
        return w * EPW + i * ECH

    def issue_in(i, slot):
        for hbm, bufs in ins:
            pltpu.async_copy(hbm.at[pl.ds(base(i), ECH)], bufs[slot], si[slot])

    def wait_in(slot):
        for hbm, bufs in ins:
            pltpu.make_async_copy(hbm.at[pl.ds(0, ECH)], bufs[slot],
                                  si[slot]).wait()

    def issue_out(i, slot):
        for hbm, obufs in outs:
            pltpu.async_copy(obufs[slot], hbm.at[pl.ds(base(i), ECH)],
                             so[slot])

    def wait_out(slot):
        for hbm, obufs in outs:
            pltpu.make_async_copy(obufs[slot], hbm.at[pl.ds(0, ECH)],
                                  so[slot]).wait()

    issue_in(0, 0)
    issue_in(1, 1)

    def body(c, carry):
        wait_in(0)
        if outs:
            @pl.when(c > 0)
            def _():
                wait_out(0)
        compute(0)
        if outs:
            issue_out(2 * c, 0)

        @pl.when(c < HALF)
        def _():
            issue_in(2 * c + 2, 0)

        wait_in(1)
        if outs:
            @pl.when(c > 0)
            def _():
                wait_out(1)
        compute(1)
        if outs:
            issue_out(2 * c + 1, 1)

        @pl.when(c < HALF - 1)
        def _():
            issue_in(2 * c + 3, 1)

        return carry

    lax.fori_loop(0, HALF, body, None)

    wait_in(0)
    if outs:
        wait_out(0)
    compute(0)
    if outs:
        issue_out(NCHUNK - 1, 0)
        wait_out(0)
        wait_out(1)


def _sc1(dst_hbm, degp_hbm, table_v, i0, i1, si0, si1, so0, so1):
    w = _wid()
    ones = jnp.ones((16,), jnp.float32)
    ib = (i0, i1)

    _zero_table2d(table_v)

    def compute(slot):
        buf = ib[slot]

        @plsc.parallel_loop(0, NCH, unroll=UNROLL)
        def _(j):
            _sca(table_v, buf[pl.ds(j * 16, 16)], ones)

    _pipe(w, [(dst_hbm, ib)], [], compute, (si0, si1), (so0, so1))
    pltpu.sync_copy(table_v, degp_hbm.at[w])


_sc1_call = pl.kernel(
    _sc1,
    out_type=jax.ShapeDtypeStruct((NW, SROW, 128), jnp.float32),
    mesh=_mesh,
    scratch_types=[
        pltpu.VMEM((SROW, 128), jnp.float32),
        pltpu.VMEM((ECH,), jnp.int32),
        pltpu.VMEM((ECH,), jnp.int32),
        pltpu.SemaphoreType.DMA,
        pltpu.SemaphoreType.DMA,
        pltpu.SemaphoreType.DMA,
        pltpu.SemaphoreType.DMA,
    ],
    compiler_params=_sc_params,
)


def _sc2(src_hbm, dst_hbm, dinv_hbm, gx_hbm, norm_hbm, gxs_hbm, paccp_hbm,
         table_v, i0, i1, i2, i3, f0, f1, f2, f3, si0, si1, so0, so1):
    w = _wid()
    si = (si0, si1)
    so = (so0, so1)

    pltpu.sync_copy(dinv_hbm, table_v)

    def norm_compute(slot):
        sbuf = (i0, i1)[slot]
        dbuf = (i2, i3)[slot]
        obuf = (f0, f1)[slot]

        @plsc.parallel_loop(0, NCH, unroll=UNROLL)
        def _(j):
            s = sbuf[pl.ds(j * 16, 16)]
            d = dbuf[pl.ds(j * 16, 16)]
            obuf[pl.ds(j * 16, 16)] = _gat(table_v, s) * _gat(table_v, d)

    _pipe(w, [(src_hbm, (i0, i1)), (dst_hbm, (i2, i3))],
          [(norm_hbm, (f0, f1))], norm_compute, si, so)

    pltpu.sync_copy(gx_hbm, table_v)

    def gxs_compute(slot):
        sbuf = (i0, i1)[slot]
        obuf = (f0, f1)[slot]

        @plsc.parallel_loop(0, NCH, unroll=UNROLL)
        def _(j):
            s = sbuf[pl.ds(j * 16, 16)]
            obuf[pl.ds(j * 16, 16)] = _gat(table_v, s)

    _pipe(w, [(src_hbm, (i0, i1))], [(gxs_hbm, (f0, f1))], gxs_compute,
          si, so)

    _zero_table2d(table_v)

    def pacc_compute(slot):
        dbuf = (i0, i1)[slot]
        vbuf = (f2, f3)[slot]

        @plsc.parallel_loop(0, NCH, unroll=UNROLL)
        def _(j):
            d = dbuf[pl.ds(j * 16, 16)]
            _sca(table_v, d, vbuf[pl.ds(j * 16, 16)])

    _pipe(w, [(dst_hbm, (i0, i1)), (gxs_hbm, (f2, f3))], [], pacc_compute,
          si, so)
    pltpu.sync_copy(table_v, paccp_hbm.at[w])


_sc2_call = pl.kernel(
    _sc2,
    out_type=(
        jax.ShapeDtypeStruct((E,), jnp.float32),
        jax.ShapeDtypeStruct((E,), jnp.float32),
        jax.ShapeDtypeStruct((NW, SROW, 128), jnp.float32),
    ),
    mesh=_mesh,
    scratch_types=[
        pltpu.VMEM((SROW, 128), jnp.float32),
        pltpu.VMEM((ECH,), jnp.int32),
        pltpu.VMEM((ECH,), jnp.int32),
        pltpu.VMEM((ECH,), jnp.int32),
        pltpu.VMEM((ECH,), jnp.int32),
        pltpu.VMEM((ECH,), jnp.float32),
        pltpu.VMEM((ECH,), jnp.float32),
        pltpu.VMEM((ECH,), jnp.float32),
        pltpu.VMEM((ECH,), jnp.float32),
        pltpu.SemaphoreType.DMA,
        pltpu.SemaphoreType.DMA,
        pltpu.SemaphoreType.DMA,
        pltpu.SemaphoreType.DMA,
    ],
    compiler_params=_sc_params,
)


def _sc3(src_hbm, dst_hbm, gs_hbm, gss_hbm, uaccp_hbm, vaccp_hbm,
         table_v, i0, i1, f0, f1, f2, f3, si0, si1, so0, so1):
    w = _wid()
    si = (si0, si1)
    so = (so0, so1)

    pltpu.sync_copy(gs_hbm, table_v)

    def gather_compute(slot):
        sbuf = (i0, i1)[slot]
        obuf = (f0, f1)[slot]

        @plsc.parallel_loop(0, NCH, unroll=UNROLL)
        def _(j):
            s = sbuf[pl.ds(j * 16, 16)]
            obuf[pl.ds(j * 16, 16)] = _gat(table_v, s)

    _pipe(w, [(src_hbm, (i0, i1))], [(gss_hbm, (f0, f1))], gather_compute,
          si, so)

    def scatter_phase(sign, out_hbm):
        _zero_table2d(table_v)

        def compute(slot):
            dbuf = (i0, i1)[slot]
            vbuf = (f2, f3)[slot]

            @plsc.parallel_loop(0, NCH, unroll=UNROLL)
            def _(j):
                d = dbuf[pl.ds(j * 16, 16)]
                v = jnp.maximum(sign * vbuf[pl.ds(j * 16, 16)], 0.0)
                _sca(table_v, d, v)

        _pipe(w, [(dst_hbm, (i0, i1)), (gss_hbm, (f2, f3))], [], compute,
              si, so)
        pltpu.sync_copy(table_v, out_hbm.at[w])

    scatter_phase(1.0, uaccp_hbm)
    scatter_phase(-1.0, vaccp_hbm)


_sc3_call = pl.kernel(
    _sc3,
    out_type=(
        jax.ShapeDtypeStruct((E,), jnp.float32),
        jax.ShapeDtypeStruct((NW, SROW, 128), jnp.float32),
        jax.ShapeDtypeStruct((NW, SROW, 128), jnp.float32),
    ),
    mesh=_mesh,
    scratch_types=[
        pltpu.VMEM((SROW, 128), jnp.float32),
        pltpu.VMEM((ECH,), jnp.int32),
        pltpu.VMEM((ECH,), jnp.int32),
        pltpu.VMEM((ECH,), jnp.float32),
        pltpu.VMEM((ECH,), jnp.float32),
        pltpu.VMEM((ECH,), jnp.float32),
        pltpu.VMEM((ECH,), jnp.float32),
        pltpu.SemaphoreType.DMA,
        pltpu.SemaphoreType.DMA,
        pltpu.SemaphoreType.DMA,
        pltpu.SemaphoreType.DMA,
    ],
    compiler_params=_sc_params,
)


def _sc4(src_hbm, dst_hbm, t_hbm, pk_hbm, w_hbm, norm_hbm,
         flat_hbm, sup_hbm, svp_hbm, cntp_hbm,
         table_v, i0, i1, i2, i3, f0, f1, f2, f3, nbt, nbp, nbw,
         su_v, sv_v, cnt_v, si0, si1, so0, so1):
    w = _wid()
    si = (si0, si1)
    so = (so0, so1)

    pltpu.sync_copy(t_hbm, table_v)

    def flat_compute(slot):
        sbuf = (i0, i1)[slot]
        dbuf = (i2, i3)[slot]
        obuf = (f0, f1)[slot]

        @plsc.parallel_loop(0, NCH, unroll=UNROLL)
        def _(j):
            s = sbuf[pl.ds(j * 16, 16)]
            d = dbuf[pl.ds(j * 16, 16)]
            ts_ = _gat(table_v, s).astype(jnp.int32)
            td = _gat(table_v, d).astype(jnp.int32)
            flat = (td >> 8) * NB + (ts_ & 255)
            obuf[pl.ds(j * 16, 16)] = flat.astype(jnp.float32)

    _pipe(w, [(src_hbm, (i0, i1)), (dst_hbm, (i2, i3))],
          [(flat_hbm, (f0, f1))], flat_compute, si, so)

    _zero_table(su_v, SUSZ)
    _zero_table(sv_v, SUSZ)
    _zero_table(cnt_v, 80)

    pltpu.sync_copy(pk_hbm, table_v)
    himask = jnp.full((16,), -65536, jnp.int32)

    def fold_compute(slot):
        sbuf = (i0, i1)[slot]
        flbuf = (f0, f1)[slot]
        nmbuf = (f2, f3)[slot]

        @plsc.parallel_loop(0, NCH, unroll=UNROLL)
        def _(j):
            s = sbuf[pl.ds(j * 16, 16)]
            g = plsc.bitcast(_gat(table_v, s), jnp.int32)
            pu_s = plsc.bitcast(g & himask, jnp.float32)
            pv_s = plsc.bitcast(g << 16, jnp.float32)
            fl = flbuf[pl.ds(j * 16, 16)].astype(jnp.int32)
            nm = nmbuf[pl.ds(j * 16, 16)]
            plsc.addupdate_scatter(su_v, [fl], nm * pu_s)
            plsc.addupdate_scatter(sv_v, [fl], nm * pv_s)

    _pipe(w, [(src_hbm, (i0, i1)), (flat_hbm, (f0, f1)),
              (norm_hbm, (f2, f3))], [], fold_compute, si, so)

    ones = jnp.ones((16,), jnp.float32)

    def do_rows(rb):
        pltpu.sync_copy(t_hbm.at[pl.ds(rb, 8)], nbt)
        pltpu.sync_copy(pk_hbm.at[pl.ds(rb, 8)], nbp)
        pltpu.sync_copy(w_hbm.at[pl.ds(rb, 8)], nbw)

        @plsc.parallel_loop(0, 64, unroll=UNROLL)
        def _(j):
            r = j >> 3
            off = (j & 7) * 16
            t = nbt[r, pl.ds(off, 16)].astype(jnp.int32)
            sg = t >> 8
            fl = sg * NB + (t & 255)
            g = plsc.bitcast(nbp[r, pl.ds(off, 16)], jnp.int32)
            pu_s = plsc.bitcast(g & himask, jnp.float32)
            pv_s = plsc.bitcast(g << 16, jnp.float32)
            wt = nbw[r, pl.ds(off, 16)]
            plsc.addupdate_scatter(su_v, [fl], wt * pu_s)
            plsc.addupdate_scatter(sv_v, [fl], wt * pv_s)
            plsc.addupdate_scatter(cnt_v, [sg], ones)

    def node_chunk(c, carry):
        do_rows((w + c * 32) * 8)
        return carry

    lax.fori_loop(0, 3, node_chunk, None)

    @pl.when(w < 2)
    def _():
        do_rows((96 + w) * 8)

    pltpu.sync_copy(su_v, sup_hbm.at[w])
    pltpu.sync_copy(sv_v, svp_hbm.at[w])
    pltpu.sync_copy(cnt_v, cntp_hbm.at[w])


_sc4_call = pl.kernel(
    _sc4,
    out_type=(
        jax.ShapeDtypeStruct((E,), jnp.float32),
        jax.ShapeDtypeStruct((NW, SUSZ), jnp.float32),
        jax.ShapeDtypeStruct((NW, SUSZ), jnp.float32),
        jax.ShapeDtypeStruct((NW, 80), jnp.float32),
    ),
    mesh=_mesh,
    scratch_types=[
        pltpu.VMEM((SROW, 128), jnp.float32),
        pltpu.VMEM((ECH,), jnp.int32),
        pltpu.VMEM((ECH,), jnp.int32),
        pltpu.VMEM((ECH,), jnp.int32),
        pltpu.VMEM((ECH,), jnp.int32),
        pltpu.VMEM((ECH,), jnp.float32),
        pltpu.VMEM((ECH,), jnp.float32),
        pltpu.VMEM((ECH,), jnp.float32),
        pltpu.VMEM((ECH,), jnp.float32),
        pltpu.VMEM((8, 128), jnp.float32),
        pltpu.VMEM((8, 128), jnp.float32),
        pltpu.VMEM((8, 128), jnp.float32),
        pltpu.VMEM((SUSZ,), jnp.float32),
        pltpu.VMEM((SUSZ,), jnp.float32),
        pltpu.VMEM((80,), jnp.float32),
        pltpu.SemaphoreType.DMA,
        pltpu.SemaphoreType.DMA,
        pltpu.SemaphoreType.DMA,
        pltpu.SemaphoreType.DMA,
    ],
    compiler_params=_sc_params,
)


def _tc1(degp_ref, x_ref, dinv_ref, gx_ref, w_ref):
    deg = jnp.sum(degp_ref[...], axis=0) + 1.0
    dinv = lax.rsqrt(deg)
    dinv_ref[...] = dinv
    gx_ref[...] = dinv * x_ref[...]
    w_ref[...] = dinv * dinv


def _tc2(paccp_ref, gx_ref, dinv_ref, gs_ref):
    acc = jnp.sum(paccp_ref[...], axis=0)
    dinv = dinv_ref[...]
    p = dinv * (acc + gx_ref[...])
    gs_ref[...] = dinv * p


def _tc3(uaccp_ref, vaccp_ref, gs_ref, dinv_ref, batchf_ref, ts_ref,
         pu_ref, pv_ref, t_ref, pk_ref):
    dinv = dinv_ref[...]
    gs = gs_ref[...]
    pu = dinv * (jnp.sum(uaccp_ref[...], axis=0) + jnp.maximum(gs, 0.0))
    pv = dinv * (jnp.sum(vaccp_ref[...], axis=0) + jnp.maximum(-gs, 0.0))
    pu_ref[...] = pu
    pv_ref[...] = pv
    r = pv / pu
    bk = jnp.sum((ts_ref[...][0][None, None, :] < r[:, :, None]).astype(jnp.float32),
                 axis=-1)
    t_ref[...] = batchf_ref[...] * 256.0 + bk
    au = lax.bitcast_convert_type(pu, jnp.int32)
    av = lax.bitcast_convert_type(pv, jnp.int32)
    pk = (au & jnp.int32(-65536)) | lax.shift_right_logical(av, 16)
    pk_ref[...] = lax.bitcast_convert_type(pk, jnp.float32)


def _tc4(sup_ref, svp_ref, cntp_ref, m_ref, al_ref, ga_ref, w3_ref, b3_ref,
         wl_ref, bl_ref, ds_ref, sw_ref, wl1_ref, bl1_ref, wl2_ref, bl2_ref,
         out_ref):
    su = jnp.sum(sup_ref[...], axis=0)
    sv = jnp.sum(svp_ref[...], axis=0)
    cnt = jnp.maximum(jnp.sum(cntp_ref[...], axis=0), 1.0)
    m = m_ref[...]
    a_mat = (al_ref[...] * jnp.dot(su, m, preferred_element_type=jnp.float32)
             + ga_ref[...] * jnp.dot(sv, m, preferred_element_type=jnp.float32))
    pooled = jnp.dot(a_mat / cnt[:, None], w3_ref[...],
                     preferred_element_type=jnp.float32) + b3_ref[...]
    z = jnp.dot(pooled, wl_ref[...],
                preferred_element_type=jnp.float32) + bl_ref[...]
    z = jnp.concatenate([z, ds_ref[...], sw_ref[...]], axis=1)
    z = jnp.maximum(jnp.dot(z, wl1_ref[...],
                            preferred_element_type=jnp.float32) + bl1_ref[...], 0.0)
    out_ref[...] = jnp.dot(z, wl2_ref[...],
                           preferred_element_type=jnp.float32) + bl2_ref[...]


def _tc_call(fn, out_shapes, **kw):
    return pl.pallas_call(fn, out_shape=out_shapes, **kw)


def kernel(x, edge_index, batch, dist, sw, W1, b1, W2, b2, W3, b3,
           Wlin, blin, Wlin1, blin1, Wlin2, blin2):
    f32 = jnp.float32
    src = edge_index[0]
    dst = edge_index[1]
    batchf = jnp.full((NPAD,), 64.0, f32).at[:N].set(batch.astype(f32))
    xpad = jnp.zeros((NPAD,), f32).at[:N].set(x[:, 0])

    a = jnp.maximum(W1[0], 0.0)
    c = jnp.maximum(-W1[0], 0.0)
    alpha = a @ W2
    gamma = c @ W2
    ts = jnp.sort(jnp.where(gamma != 0, -alpha / jnp.where(gamma != 0, gamma, 1.0), 0.0))
    r_rep = jnp.concatenate([ts[:1] - 1.0, (ts[:-1] + ts[1:]) * 0.5, ts[-1:] + 1.0])
    M = ((alpha[None, :] + gamma[None, :] * r_rep[:, None]) > 0.0).astype(f32)

    degp = _sc1_call(dst)

    nshape = jax.ShapeDtypeStruct((SROW, 128), f32)
    dinv2d, gx2d, w2d = _tc_call(_tc1, (nshape, nshape, nshape))(
        degp, xpad.reshape(SROW, 128))

    norm, gxs, paccp = _sc2_call(src, dst, dinv2d, gx2d)

    gs2d = _tc_call(_tc2, nshape)(paccp, gx2d, dinv2d)

    _, uaccp, vaccp = _sc3_call(src, dst, gs2d)

    grid = SROW // 8
    blk_n = pl.BlockSpec((8, 128), lambda i: (i, 0))
    blk_p = pl.BlockSpec((NW, 8, 128), lambda i: (0, i, 0))
    blk_ts = pl.BlockSpec((1, 64), lambda i: (0, 0))
    pu2d, pv2d, t2d, pk2d = _tc_call(
        _tc3, (nshape, nshape, nshape, nshape),
        grid=(grid,),
        in_specs=[blk_p, blk_p, blk_n, blk_n, blk_n, blk_ts],
        out_specs=(blk_n, blk_n, blk_n, blk_n),
    )(uaccp, vaccp,
      gs2d, dinv2d, batchf.reshape(SROW, 128), ts[None, :])

    _, sup, svp, cntp = _sc4_call(src, dst, t2d, pk2d, w2d, norm)

    out = _tc_call(_tc4, jax.ShapeDtypeStruct((B, 1), f32))(
        sup[:, :B * NB].reshape(NW, B, NB), svp[:, :B * NB].reshape(NW, B, NB),
        cntp[:, :B], M, alpha[None, :], gamma[None, :], W3, b3[None, :],
        Wlin, blin[None, :], dist, sw, Wlin1, blin1[None, :], Wlin2,
        blin2[None, :])
    return out
